# Initial kernel scaffold; baseline (speedup 1.0000x reference)
import functools

import jax
import jax.numpy as jnp
from jax import lax
from jax.experimental import pallas as pl
from jax.experimental.pallas import tpu as pltpu
from jax.experimental.pallas import tpu_sc as plsc

N = 10000
E = 160000
D = 256
DH = 128
HID = 512
SLOPE = 0.2

LANES = 16
NSUB = 16
CH = 128
EPW = 10112
EPAD = NSUB * EPW
NPAD = 10240
STRIPE = NPAD // NSUB


def _lk(x):
    return jnp.where(x >= 0, x, SLOPE * x)


def _dot(a, b):
    return jnp.dot(a, b, preferred_element_type=jnp.float32)


def _features_tc(feats, fu_ws, fu_bs, proj_ws, bsum, rows_per_blk=1000):
    nf = len(feats)
    grid = (N // rows_per_blk,)

    def body(*refs):
        frefs = refs[:nf]
        fw = refs[nf:2 * nf]
        fb = refs[2 * nf:3 * nf]
        pw = refs[3 * nf:4 * nf]
        bsum_r = refs[4 * nf]
        out_ref = refs[4 * nf + 1]
        acc = None
        for i in range(nf):
            u = _lk(_dot(frefs[i][...], fw[i][...]) + fb[i][...])
            t = _dot(u, pw[i][...])
            acc = t if acc is None else acc + t
        x = acc + bsum_r[...]
        out_ref[0] = x[:, :DH]
        out_ref[1] = x[:, DH:]

    in_specs = []
    for f in feats:
        d = f.shape[1]
        in_specs.append(pl.BlockSpec((rows_per_blk, d), lambda i: (i, 0)))
    for w in fu_ws:
        in_specs.append(pl.BlockSpec(w.shape, lambda i: (0, 0)))
    for b in fu_bs:
        in_specs.append(pl.BlockSpec(b.shape, lambda i: (0, 0)))
    for w in proj_ws:
        in_specs.append(pl.BlockSpec(w.shape, lambda i: (0, 0)))
    in_specs.append(pl.BlockSpec(bsum.shape, lambda i: (0, 0)))

    return pl.pallas_call(
        body,
        grid=grid,
        in_specs=in_specs,
        out_specs=pl.BlockSpec((2, rows_per_blk, DH), lambda i: (0, i, 0)),
        out_shape=jax.ShapeDtypeStruct((2, N, DH), jnp.float32),
    )(*feats, *fu_ws, *fu_bs, *proj_ws, bsum)


_SC_MESH = plsc.VectorSubcoreMesh(core_axis_name="c", subcore_axis_name="s")


@functools.partial(
    pl.kernel,
    mesh=_SC_MESH,
    out_type=jax.ShapeDtypeStruct((2 * NPAD, DH), jnp.float32),
    scratch_types=[
        pltpu.VMEM((CH,), jnp.int32),
        pltpu.VMEM((CH,), jnp.int32),
        pltpu.VMEM((CH, DH), jnp.float32),
        pltpu.VMEM((CH, LANES), jnp.float32),
        pltpu.VMEM((STRIPE, DH), jnp.float32),
        pltpu.VMEM((STRIPE, LANES), jnp.float32),
        pltpu.VMEM_SHARED((NPAD, DH), jnp.float32),
        pltpu.VMEM_SHARED((NPAD, LANES), jnp.float32),
        pltpu.SemaphoreType.DMA,
    ],
)
def _seg_mean_sc(table_h, src_h, dst_h, out_h,
                 src_v, dst_v, rows_v, ones_v, stage_v, cstage_v,
                 acc_sh, cnt_sh, sem):
    c = lax.axis_index("c")
    s = lax.axis_index("s")
    zero16 = jnp.zeros((LANES,), jnp.float32)
    one16 = jnp.ones((LANES,), jnp.float32)

    def zrow(r, carry):
        for j in range(DH // LANES):
            stage_v[r, pl.ds(j * LANES, LANES)] = zero16
        cstage_v[r, :] = zero16
        return carry

    lax.fori_loop(0, STRIPE, zrow, 0)

    def orow(r, carry):
        ones_v[r, :] = one16
        return carry

    lax.fori_loop(0, CH, orow, 0)

    row0 = s * STRIPE
    pltpu.sync_copy(stage_v, acc_sh.at[pl.ds(row0, STRIPE)])
    pltpu.sync_copy(cstage_v, cnt_sh.at[pl.ds(row0, STRIPE)])
    plsc.subcore_barrier()

    ebase = s * EPW
    sbase = c * EPAD + ebase

    def chunk(i, carry):
        pltpu.sync_copy(src_h.at[pl.ds(sbase + i * CH, CH)], src_v)
        pltpu.sync_copy(dst_h.at[pl.ds(ebase + i * CH, CH)], dst_v)
        pltpu.async_copy(table_h.at[src_v], rows_v, sem).wait()
        pltpu.sync_copy(rows_v, acc_sh.at[dst_v], add=True)
        pltpu.sync_copy(ones_v, cnt_sh.at[dst_v], add=True)
        return carry

    lax.fori_loop(0, EPW // CH, chunk, 0)
    plsc.subcore_barrier()

    pltpu.sync_copy(acc_sh.at[pl.ds(row0, STRIPE)], stage_v)
    pltpu.sync_copy(cnt_sh.at[pl.ds(row0, STRIPE)], cstage_v)

    def div_row(r, carry):
        inv = 1.0 / jnp.maximum(cstage_v[r, :], 1.0)
        for j in range(DH // LANES):
            sl = pl.ds(j * LANES, LANES)
            stage_v[r, sl] = stage_v[r, sl] * inv
        return carry

    lax.fori_loop(0, STRIPE, div_row, 0)
    pltpu.sync_copy(stage_v, out_h.at[pl.ds(c * NPAD + row0, STRIPE)])


def _layer1_tc(a0, a1, x_split, Wl, bl, Wr0, Wr1, W2l, rows_per_blk=1000):
    grid = (N // rows_per_blk,)

    def body(a0_r, a1_r, xs_r, Wl_r, bl_r, Wr0_r, Wr1_r, W2_r, h_ref, y2_ref):
        mean = jnp.concatenate([a0_r[...], a1_r[...]], axis=1)
        pre = (_dot(mean, Wl_r[...]) + bl_r[...]
               + _dot(xs_r[0], Wr0_r[...]) + _dot(xs_r[1], Wr1_r[...]))
        h = _lk(pre)
        h_ref[...] = h
        y2 = _dot(h, W2_r[...])
        y2_ref[0] = y2[:, :DH]
        y2_ref[1] = y2[:, DH:]

    R = rows_per_blk
    return pl.pallas_call(
        body,
        grid=grid,
        in_specs=[
            pl.BlockSpec((R, DH), lambda i: (i, 0)),
            pl.BlockSpec((R, DH), lambda i: (i, 0)),
            pl.BlockSpec((2, R, DH), lambda i: (0, i, 0)),
            pl.BlockSpec(Wl.shape, lambda i: (0, 0)),
            pl.BlockSpec(bl.shape, lambda i: (0, 0)),
            pl.BlockSpec(Wr0.shape, lambda i: (0, 0)),
            pl.BlockSpec(Wr1.shape, lambda i: (0, 0)),
            pl.BlockSpec(W2l.shape, lambda i: (0, 0)),
        ],
        out_specs=[
            pl.BlockSpec((R, HID), lambda i: (i, 0)),
            pl.BlockSpec((2, R, DH), lambda i: (0, i, 0)),
        ],
        out_shape=[
            jax.ShapeDtypeStruct((N, HID), jnp.float32),
            jax.ShapeDtypeStruct((2, N, DH), jnp.float32),
        ],
    )(a0, a1, x_split, Wl, bl, Wr0, Wr1, W2l)


def _layer2_tc(b0, b1, h, Wr, bl, rows_per_blk=1000):
    grid = (N // rows_per_blk,)

    def body(b0_r, b1_r, h_r, Wr_r, bl_r, out_ref):
        mean = jnp.concatenate([b0_r[...], b1_r[...]], axis=1)
        out_ref[...] = mean + bl_r[...] + _dot(h_r[...], Wr_r[...])

    R = rows_per_blk
    return pl.pallas_call(
        body,
        grid=grid,
        in_specs=[
            pl.BlockSpec((R, DH), lambda i: (i, 0)),
            pl.BlockSpec((R, DH), lambda i: (i, 0)),
            pl.BlockSpec((R, HID), lambda i: (i, 0)),
            pl.BlockSpec(Wr.shape, lambda i: (0, 0)),
            pl.BlockSpec(bl.shape, lambda i: (0, 0)),
        ],
        out_specs=pl.BlockSpec((R, D), lambda i: (i, 0)),
        out_shape=jax.ShapeDtypeStruct((N, D), jnp.float32),
    )(b0, b1, h, Wr, bl)


def _pad_edges(ei):
    pad = EPAD - E
    src = jnp.concatenate([ei[0].astype(jnp.int32),
                           jnp.zeros((pad,), jnp.int32)])
    dst = jnp.concatenate([ei[1].astype(jnp.int32),
                           jnp.full((pad,), N, jnp.int32)])
    srcs2 = jnp.concatenate([src, src + N])
    return srcs2, dst


def _halves(seg_out):
    return seg_out[:N], seg_out[NPAD:NPAD + N]


def kernel(m_emb_feat, m_sim_feat, m_ass_feat, d_sim_feat, d_ass_feat,
           ei_md, ei_dm,
           fu_m_emb_W, fu_m_emb_b, fu_m_sim_W, fu_m_sim_b,
           fu_m_ass_W, fu_m_ass_b, fu_d_sim_W, fu_d_sim_b,
           fu_d_ass_W, fu_d_ass_b,
           m_emb_W, m_emb_b, m_sim_W, m_sim_b, m_ass_W, m_ass_b,
           d_sim_W, d_sim_b, d_ass_W, d_ass_b,
           w_m_emb, w_m_sim, w_m_ass, w_d_sim, w_d_ass,
           l1_md_Wl, l1_md_bl, l1_md_Wr,
           l2_md_Wl, l2_md_bl, l2_md_Wr,
           l1_dm_Wl, l1_dm_bl, l1_dm_Wr,
           l2_dm_Wl, l2_dm_bl, l2_dm_Wr):
    pm = [m_emb_W * w_m_emb, m_sim_W * w_m_sim, m_ass_W * w_m_ass]
    bm = (m_emb_b * w_m_emb + m_sim_b * w_m_sim
          + m_ass_b * w_m_ass).reshape(1, D)
    pd = [d_sim_W * w_d_sim, d_ass_W * w_d_ass]
    bd = (d_sim_b * w_d_sim + d_ass_b * w_d_ass).reshape(1, D)

    x_m = _features_tc(
        [m_emb_feat, m_sim_feat, m_ass_feat],
        [fu_m_emb_W, fu_m_sim_W, fu_m_ass_W],
        [fu_m_emb_b.reshape(1, -1), fu_m_sim_b.reshape(1, -1),
         fu_m_ass_b.reshape(1, -1)],
        pm, bm)
    x_d = _features_tc(
        [d_sim_feat, d_ass_feat],
        [fu_d_sim_W, fu_d_ass_W],
        [fu_d_sim_b.reshape(1, -1), fu_d_ass_b.reshape(1, -1)],
        pd, bd)

    src_md, dst_md = _pad_edges(ei_md)
    src_dm, dst_dm = _pad_edges(ei_dm)

    mean_md = _seg_mean_sc(x_m.reshape(2 * N, DH), src_md, dst_md)
    mean_dm = _seg_mean_sc(x_d.reshape(2 * N, DH), src_dm, dst_dm)

    a0_md, a1_md = _halves(mean_md)
    a0_dm, a1_dm = _halves(mean_dm)

    h_d, y2_dm = _layer1_tc(a0_md, a1_md, x_d,
                            l1_md_Wl, l1_md_bl.reshape(1, -1),
                            l1_md_Wr[:DH], l1_md_Wr[DH:], l2_dm_Wl)
    h_m, y2_md = _layer1_tc(a0_dm, a1_dm, x_m,
                            l1_dm_Wl, l1_dm_bl.reshape(1, -1),
                            l1_dm_Wr[:DH], l1_dm_Wr[DH:], l2_md_Wl)

    mean2_md = _seg_mean_sc(y2_md.reshape(2 * N, DH), src_md, dst_md)
    mean2_dm = _seg_mean_sc(y2_dm.reshape(2 * N, DH), src_dm, dst_dm)

    b0_md, b1_md = _halves(mean2_md)
    b0_dm, b1_dm = _halves(mean2_dm)

    out_d = _layer2_tc(b0_md, b1_md, h_d, l2_md_Wr, l2_md_bl.reshape(1, -1))
    out_m = _layer2_tc(b0_dm, b1_dm, h_m, l2_dm_Wr, l2_dm_bl.reshape(1, -1))
    return jnp.concatenate([out_m, out_d], axis=0)


# speedup vs baseline: 3.0609x; 3.0609x over previous
import functools

import jax
import jax.numpy as jnp
from jax import lax
from jax.experimental import pallas as pl
from jax.experimental.pallas import tpu as pltpu
from jax.experimental.pallas import tpu_sc as plsc

N = 10000
E = 160000
D = 256
DH = 128
HID = 512
SLOPE = 0.2

LANES = 16
NSUB = 16
CH = 128
EPW = 10112
EPAD = NSUB * EPW
NPAD = 10240
STRIPE = NPAD // NSUB
CH2 = 64
EPW2 = EPAD // 32
WCH = 64


def _lk(x):
    return jnp.where(x >= 0, x, SLOPE * x)


def _dot(a, b):
    return jnp.dot(a, b, preferred_element_type=jnp.float32)


def _features_tc(feats, fu_ws, fu_bs, proj_ws, bsum, rows_per_blk=1000):
    nf = len(feats)
    grid = (N // rows_per_blk,)

    def body(*refs):
        frefs = refs[:nf]
        fw = refs[nf:2 * nf]
        fb = refs[2 * nf:3 * nf]
        pw = refs[3 * nf:4 * nf]
        bsum_r = refs[4 * nf]
        out_ref = refs[4 * nf + 1]
        acc = None
        for i in range(nf):
            u = _lk(_dot(frefs[i][...], fw[i][...]) + fb[i][...])
            t = _dot(u, pw[i][...])
            acc = t if acc is None else acc + t
        x = acc + bsum_r[...]
        out_ref[0] = x[:, :DH]
        out_ref[1] = x[:, DH:]

    in_specs = []
    for f in feats:
        d = f.shape[1]
        in_specs.append(pl.BlockSpec((rows_per_blk, d), lambda i: (i, 0)))
    for w in fu_ws:
        in_specs.append(pl.BlockSpec(w.shape, lambda i: (0, 0)))
    for b in fu_bs:
        in_specs.append(pl.BlockSpec(b.shape, lambda i: (0, 0)))
    for w in proj_ws:
        in_specs.append(pl.BlockSpec(w.shape, lambda i: (0, 0)))
    in_specs.append(pl.BlockSpec(bsum.shape, lambda i: (0, 0)))

    return pl.pallas_call(
        body,
        grid=grid,
        in_specs=in_specs,
        out_specs=pl.BlockSpec((2, rows_per_blk, DH), lambda i: (0, i, 0)),
        out_shape=jax.ShapeDtypeStruct((2, N, DH), jnp.float32),
    )(*feats, *fu_ws, *fu_bs, *proj_ws, bsum)


def _cnt_body(dst_h, out_h, dst_v, ones_v, cstage_v, cnt_sh):
    c = lax.axis_index("c")
    s = lax.axis_index("s")
    zero16 = jnp.zeros((LANES,), jnp.float32)
    one16 = jnp.ones((LANES,), jnp.float32)

    def zrow(r, carry):
        for j in range(DH // LANES):
            cstage_v[r, pl.ds(j * LANES, LANES)] = zero16
        return carry

    lax.fori_loop(0, WCH, zrow, 0)

    row0 = s * STRIPE

    def zchunk(k, carry):
        pltpu.sync_copy(cstage_v, cnt_sh.at[pl.ds(row0 + k * WCH, WCH)])
        return carry

    lax.fori_loop(0, STRIPE // WCH, zchunk, 0)

    def orow(r, carry):
        for j in range(DH // LANES):
            ones_v[r, pl.ds(j * LANES, LANES)] = one16
        return carry

    lax.fori_loop(0, CH2, orow, 0)
    plsc.subcore_barrier()

    ebase = c * (EPAD // 2) + s * EPW2

    def chunk(i, carry):
        pltpu.sync_copy(dst_h.at[pl.ds(ebase + i * CH2, CH2)], dst_v)
        pltpu.sync_copy(ones_v, cnt_sh.at[dst_v], add=True)
        return carry

    lax.fori_loop(0, EPW2 // CH2, chunk, 0)
    plsc.subcore_barrier()

    def wchunk(k, carry):
        r0 = row0 + k * WCH
        pltpu.sync_copy(cnt_sh.at[pl.ds(r0, WCH)], cstage_v)
        pltpu.sync_copy(cstage_v, out_h.at[pl.ds(c * NPAD + r0, WCH)])
        return carry

    lax.fori_loop(0, STRIPE // WCH, wchunk, 0)


def _seg_body(table_h, src_h, dst_h, inv_h, out_h,
              src_v, dst_v, rows_v, stage_v, cstage_v,
              acc_sh, sem):
    c = lax.axis_index("c")
    s = lax.axis_index("s")
    zero16 = jnp.zeros((LANES,), jnp.float32)

    def zrow(r, carry):
        for j in range(DH // LANES):
            stage_v[r, pl.ds(j * LANES, LANES)] = zero16
        return carry

    lax.fori_loop(0, WCH, zrow, 0)

    row0 = s * STRIPE

    def zchunk(k, carry):
        pltpu.sync_copy(stage_v, acc_sh.at[pl.ds(row0 + k * WCH, WCH)])
        return carry

    lax.fori_loop(0, STRIPE // WCH, zchunk, 0)
    plsc.subcore_barrier()

    ebase = s * EPW
    sbase = c * EPAD + ebase

    def chunk(i, carry):
        pltpu.sync_copy(src_h.at[pl.ds(sbase + i * CH, CH)], src_v)
        pltpu.sync_copy(dst_h.at[pl.ds(ebase + i * CH, CH)], dst_v)
        pltpu.async_copy(table_h.at[src_v], rows_v, sem).wait()
        pltpu.sync_copy(rows_v, acc_sh.at[dst_v], add=True)
        return carry

    lax.fori_loop(0, EPW // CH, chunk, 0)
    plsc.subcore_barrier()

    def wchunk(k, carry):
        r0 = row0 + k * WCH
        pltpu.sync_copy(acc_sh.at[pl.ds(r0, WCH)], stage_v)
        pltpu.sync_copy(inv_h.at[pl.ds(r0, WCH)], cstage_v)

        def div_row(r, carry2):
            inv = cstage_v[r, :]
            for j in range(DH // LANES):
                sl = pl.ds(j * LANES, LANES)
                stage_v[r, sl] = stage_v[r, sl] * inv
            return carry2

        lax.fori_loop(0, WCH, div_row, 0)
        pltpu.sync_copy(stage_v, out_h.at[pl.ds(c * NPAD + r0, WCH)])
        return carry

    lax.fori_loop(0, STRIPE // WCH, wchunk, 0)


_SC_CACHE = {}


def _sc_mesh():
    return plsc.VectorSubcoreMesh(core_axis_name="c", subcore_axis_name="s")


def _cnt_sc(dst):
    if "cnt" not in _SC_CACHE:
        _SC_CACHE["cnt"] = functools.partial(
            pl.kernel,
            mesh=_sc_mesh(),
            out_type=jax.ShapeDtypeStruct((2 * NPAD, DH), jnp.float32),
            scratch_types=[
                pltpu.VMEM((CH2,), jnp.int32),
                pltpu.VMEM((CH2, DH), jnp.float32),
                pltpu.VMEM((WCH, DH), jnp.float32),
                pltpu.VMEM_SHARED((NPAD, DH), jnp.float32),
            ],
        )(_cnt_body)
    return _SC_CACHE["cnt"](dst)


def _seg_mean_sc(table, srcs2, dst, inv):
    if "seg" not in _SC_CACHE:
        _SC_CACHE["seg"] = functools.partial(
            pl.kernel,
            mesh=_sc_mesh(),
            out_type=jax.ShapeDtypeStruct((2 * NPAD, DH), jnp.float32),
            scratch_types=[
                pltpu.VMEM((CH,), jnp.int32),
                pltpu.VMEM((CH,), jnp.int32),
                pltpu.VMEM((CH, DH), jnp.float32),
                pltpu.VMEM((WCH, DH), jnp.float32),
                pltpu.VMEM((WCH, LANES), jnp.float32),
                pltpu.VMEM_SHARED((NPAD, DH), jnp.float32),
                pltpu.SemaphoreType.DMA,
            ],
        )(_seg_body)
    return _SC_CACHE["seg"](table, srcs2, dst, inv)


def _layer1_tc(a0, a1, x_split, Wl, bl, Wr0, Wr1, W2l, rows_per_blk=1000):
    grid = (N // rows_per_blk,)

    def body(a0_r, a1_r, xs_r, Wl_r, bl_r, Wr0_r, Wr1_r, W2_r, h_ref, y2_ref):
        mean = jnp.concatenate([a0_r[...], a1_r[...]], axis=1)
        pre = (_dot(mean, Wl_r[...]) + bl_r[...]
               + _dot(xs_r[0], Wr0_r[...]) + _dot(xs_r[1], Wr1_r[...]))
        h = _lk(pre)
        h_ref[...] = h
        y2 = _dot(h, W2_r[...])
        y2_ref[0] = y2[:, :DH]
        y2_ref[1] = y2[:, DH:]

    R = rows_per_blk
    return pl.pallas_call(
        body,
        grid=grid,
        in_specs=[
            pl.BlockSpec((R, DH), lambda i: (i, 0)),
            pl.BlockSpec((R, DH), lambda i: (i, 0)),
            pl.BlockSpec((2, R, DH), lambda i: (0, i, 0)),
            pl.BlockSpec(Wl.shape, lambda i: (0, 0)),
            pl.BlockSpec(bl.shape, lambda i: (0, 0)),
            pl.BlockSpec(Wr0.shape, lambda i: (0, 0)),
            pl.BlockSpec(Wr1.shape, lambda i: (0, 0)),
            pl.BlockSpec(W2l.shape, lambda i: (0, 0)),
        ],
        out_specs=[
            pl.BlockSpec((R, HID), lambda i: (i, 0)),
            pl.BlockSpec((2, R, DH), lambda i: (0, i, 0)),
        ],
        out_shape=[
            jax.ShapeDtypeStruct((N, HID), jnp.float32),
            jax.ShapeDtypeStruct((2, N, DH), jnp.float32),
        ],
    )(a0, a1, x_split, Wl, bl, Wr0, Wr1, W2l)


def _layer2_tc(b0, b1, h, Wr, bl, rows_per_blk=1000):
    grid = (N // rows_per_blk,)

    def body(b0_r, b1_r, h_r, Wr_r, bl_r, out_ref):
        mean = jnp.concatenate([b0_r[...], b1_r[...]], axis=1)
        out_ref[...] = mean + bl_r[...] + _dot(h_r[...], Wr_r[...])

    R = rows_per_blk
    return pl.pallas_call(
        body,
        grid=grid,
        in_specs=[
            pl.BlockSpec((R, DH), lambda i: (i, 0)),
            pl.BlockSpec((R, DH), lambda i: (i, 0)),
            pl.BlockSpec((R, HID), lambda i: (i, 0)),
            pl.BlockSpec(Wr.shape, lambda i: (0, 0)),
            pl.BlockSpec(bl.shape, lambda i: (0, 0)),
        ],
        out_specs=pl.BlockSpec((R, D), lambda i: (i, 0)),
        out_shape=jax.ShapeDtypeStruct((N, D), jnp.float32),
    )(b0, b1, h, Wr, bl)


def _pad_edges(ei):
    pad = EPAD - E
    src = jnp.concatenate([ei[0].astype(jnp.int32),
                           jnp.zeros((pad,), jnp.int32)])
    dst = jnp.concatenate([ei[1].astype(jnp.int32),
                           jnp.full((pad,), N, jnp.int32)])
    srcs2 = jnp.concatenate([src, src + N])
    return srcs2, dst


def _halves(seg_out):
    return seg_out[:N], seg_out[NPAD:NPAD + N]


def kernel(m_emb_feat, m_sim_feat, m_ass_feat, d_sim_feat, d_ass_feat,
           ei_md, ei_dm,
           fu_m_emb_W, fu_m_emb_b, fu_m_sim_W, fu_m_sim_b,
           fu_m_ass_W, fu_m_ass_b, fu_d_sim_W, fu_d_sim_b,
           fu_d_ass_W, fu_d_ass_b,
           m_emb_W, m_emb_b, m_sim_W, m_sim_b, m_ass_W, m_ass_b,
           d_sim_W, d_sim_b, d_ass_W, d_ass_b,
           w_m_emb, w_m_sim, w_m_ass, w_d_sim, w_d_ass,
           l1_md_Wl, l1_md_bl, l1_md_Wr,
           l2_md_Wl, l2_md_bl, l2_md_Wr,
           l1_dm_Wl, l1_dm_bl, l1_dm_Wr,
           l2_dm_Wl, l2_dm_bl, l2_dm_Wr):
    pm = [m_emb_W * w_m_emb, m_sim_W * w_m_sim, m_ass_W * w_m_ass]
    bm = (m_emb_b * w_m_emb + m_sim_b * w_m_sim
          + m_ass_b * w_m_ass).reshape(1, D)
    pd = [d_sim_W * w_d_sim, d_ass_W * w_d_ass]
    bd = (d_sim_b * w_d_sim + d_ass_b * w_d_ass).reshape(1, D)

    x_m = _features_tc(
        [m_emb_feat, m_sim_feat, m_ass_feat],
        [fu_m_emb_W, fu_m_sim_W, fu_m_ass_W],
        [fu_m_emb_b.reshape(1, -1), fu_m_sim_b.reshape(1, -1),
         fu_m_ass_b.reshape(1, -1)],
        pm, bm)
    x_d = _features_tc(
        [d_sim_feat, d_ass_feat],
        [fu_d_sim_W, fu_d_ass_W],
        [fu_d_sim_b.reshape(1, -1), fu_d_ass_b.reshape(1, -1)],
        pd, bd)

    src_md, dst_md = _pad_edges(ei_md)
    src_dm, dst_dm = _pad_edges(ei_dm)

    craw_md = _cnt_sc(dst_md)
    craw_dm = _cnt_sc(dst_dm)

    def _inv(craw):
        cnt = craw[:NPAD, 0] + craw[NPAD:, 0]
        return jnp.broadcast_to(
            (1.0 / jnp.maximum(cnt, 1.0))[:, None], (NPAD, LANES))

    inv_md = _inv(craw_md)
    inv_dm = _inv(craw_dm)

    mean_md = _seg_mean_sc(x_m.reshape(2 * N, DH), src_md, dst_md, inv_md)
    mean_dm = _seg_mean_sc(x_d.reshape(2 * N, DH), src_dm, dst_dm, inv_dm)

    a0_md, a1_md = _halves(mean_md)
    a0_dm, a1_dm = _halves(mean_dm)

    h_d, y2_dm = _layer1_tc(a0_md, a1_md, x_d,
                            l1_md_Wl, l1_md_bl.reshape(1, -1),
                            l1_md_Wr[:DH], l1_md_Wr[DH:], l2_dm_Wl)
    h_m, y2_md = _layer1_tc(a0_dm, a1_dm, x_m,
                            l1_dm_Wl, l1_dm_bl.reshape(1, -1),
                            l1_dm_Wr[:DH], l1_dm_Wr[DH:], l2_md_Wl)

    mean2_md = _seg_mean_sc(y2_md.reshape(2 * N, DH), src_md, dst_md, inv_md)
    mean2_dm = _seg_mean_sc(y2_dm.reshape(2 * N, DH), src_dm, dst_dm, inv_dm)

    b0_md, b1_md = _halves(mean2_md)
    b0_dm, b1_dm = _halves(mean2_dm)

    out_d = _layer2_tc(b0_md, b1_md, h_d, l2_md_Wr, l2_md_bl.reshape(1, -1))
    out_m = _layer2_tc(b0_dm, b1_dm, h_m, l2_dm_Wr, l2_dm_bl.reshape(1, -1))
    return jnp.concatenate([out_m, out_d], axis=0)


# speedup vs baseline: 3.8930x; 1.2719x over previous
import functools

import jax
import jax.numpy as jnp
from jax import lax
from jax.experimental import pallas as pl
from jax.experimental.pallas import tpu as pltpu
from jax.experimental.pallas import tpu_sc as plsc

N = 10000
E = 160000
D = 256
DH = 128
HID = 512
SLOPE = 0.2

LANES = 16
NSUB = 16
CH = 128
EPW = 10112
EPAD = NSUB * EPW
NPAD = 10240
STRIPE = NPAD // NSUB
CH2 = 64
EPW2 = EPAD // 32
WCH = 32


def _lk(x):
    return jnp.where(x >= 0, x, SLOPE * x)


def _dot(a, b):
    return jnp.dot(a, b, preferred_element_type=jnp.float32)


def _features_tc(feats, fu_ws, fu_bs, proj_ws, bsum, rows_per_blk=1000):
    nf = len(feats)
    grid = (N // rows_per_blk,)

    def body(*refs):
        frefs = refs[:nf]
        fw = refs[nf:2 * nf]
        fb = refs[2 * nf:3 * nf]
        pw = refs[3 * nf:4 * nf]
        bsum_r = refs[4 * nf]
        out_ref = refs[4 * nf + 1]
        acc = None
        for i in range(nf):
            u = _lk(_dot(frefs[i][...], fw[i][...]) + fb[i][...])
            t = _dot(u, pw[i][...])
            acc = t if acc is None else acc + t
        x = acc + bsum_r[...]
        out_ref[0] = x[:, :DH]
        out_ref[1] = x[:, DH:]

    in_specs = []
    for f in feats:
        d = f.shape[1]
        in_specs.append(pl.BlockSpec((rows_per_blk, d), lambda i: (i, 0)))
    for w in fu_ws:
        in_specs.append(pl.BlockSpec(w.shape, lambda i: (0, 0)))
    for b in fu_bs:
        in_specs.append(pl.BlockSpec(b.shape, lambda i: (0, 0)))
    for w in proj_ws:
        in_specs.append(pl.BlockSpec(w.shape, lambda i: (0, 0)))
    in_specs.append(pl.BlockSpec(bsum.shape, lambda i: (0, 0)))

    return pl.pallas_call(
        body,
        grid=grid,
        in_specs=in_specs,
        out_specs=pl.BlockSpec((2, rows_per_blk, DH), lambda i: (0, i, 0)),
        out_shape=jax.ShapeDtypeStruct((2, N, DH), jnp.float32),
    )(*feats, *fu_ws, *fu_bs, *proj_ws, bsum)


def _cnt_body(dst_h, out_h, dst_v, ones_v, cstage_v, cnt_sh):
    c = lax.axis_index("c")
    s = lax.axis_index("s")
    zero16 = jnp.zeros((LANES,), jnp.float32)
    one16 = jnp.ones((LANES,), jnp.float32)

    def zrow(r, carry):
        for j in range(DH // LANES):
            cstage_v[r, pl.ds(j * LANES, LANES)] = zero16
        return carry

    lax.fori_loop(0, WCH, zrow, 0)

    row0 = s * STRIPE

    def zchunk(k, carry):
        pltpu.sync_copy(cstage_v, cnt_sh.at[pl.ds(row0 + k * WCH, WCH)])
        return carry

    lax.fori_loop(0, STRIPE // WCH, zchunk, 0)

    def orow(r, carry):
        for j in range(DH // LANES):
            ones_v[r, pl.ds(j * LANES, LANES)] = one16
        return carry

    lax.fori_loop(0, CH2, orow, 0)
    plsc.subcore_barrier()

    ebase = c * (EPAD // 2) + s * EPW2

    def chunk(i, carry):
        pltpu.sync_copy(dst_h.at[pl.ds(ebase + i * CH2, CH2)], dst_v)
        pltpu.sync_copy(ones_v, cnt_sh.at[dst_v], add=True)
        return carry

    lax.fori_loop(0, EPW2 // CH2, chunk, 0)
    plsc.subcore_barrier()

    def wchunk(k, carry):
        r0 = row0 + k * WCH
        pltpu.sync_copy(cnt_sh.at[pl.ds(r0, WCH)], cstage_v)
        pltpu.sync_copy(cstage_v, out_h.at[pl.ds(c * NPAD + r0, WCH)])
        return carry

    lax.fori_loop(0, STRIPE // WCH, wchunk, 0)


def _seg_body(table_h, src_h, dst_h, inv_h, out_h,
              src0_v, dst0_v, rows0_v, src1_v, dst1_v, rows1_v,
              stage_v, cstage_v, acc_sh, sem0, sem1):
    c = lax.axis_index("c")
    s = lax.axis_index("s")
    zero16 = jnp.zeros((LANES,), jnp.float32)

    def zrow(r, carry):
        for j in range(DH // LANES):
            stage_v[r, pl.ds(j * LANES, LANES)] = zero16
        return carry

    lax.fori_loop(0, WCH, zrow, 0)

    row0 = s * STRIPE

    def zchunk(k, carry):
        pltpu.sync_copy(stage_v, acc_sh.at[pl.ds(row0 + k * WCH, WCH)])
        return carry

    lax.fori_loop(0, STRIPE // WCH, zchunk, 0)
    plsc.subcore_barrier()

    ebase = s * EPW
    sbase = c * EPAD + ebase
    bufs = ((src0_v, dst0_v, rows0_v, sem0),
            (src1_v, dst1_v, rows1_v, sem1))

    def load_idx(j, buf):
        pltpu.sync_copy(src_h.at[pl.ds(sbase + j * CH, CH)], buf[0])
        pltpu.sync_copy(dst_h.at[pl.ds(ebase + j * CH, CH)], buf[1])

    load_idx(0, bufs[0])
    pltpu.async_copy(table_h.at[src0_v], rows0_v, sem0)

    def pair(i2, carry):
        for p in range(2):
            j = i2 * 2 + p
            cur, nxt = bufs[p], bufs[1 - p]
            load_idx(j + 1, nxt)
            pltpu.async_copy(table_h.at[nxt[0]], nxt[2], nxt[3])
            pltpu.make_async_copy(table_h.at[cur[0]], cur[2], cur[3]).wait()
            pltpu.sync_copy(cur[2], acc_sh.at[cur[1]], add=True)
        return carry

    lax.fori_loop(0, (EPW // CH) // 2, pair, 0)
    pltpu.make_async_copy(table_h.at[src0_v], rows0_v, sem0).wait()
    pltpu.sync_copy(rows0_v, acc_sh.at[dst0_v], add=True)
    plsc.subcore_barrier()

    def wchunk(k, carry):
        r0 = row0 + k * WCH
        pltpu.sync_copy(acc_sh.at[pl.ds(r0, WCH)], stage_v)
        pltpu.sync_copy(inv_h.at[pl.ds(r0, WCH)], cstage_v)

        def div_row(r, carry2):
            inv = cstage_v[r, :]
            for j in range(DH // LANES):
                sl = pl.ds(j * LANES, LANES)
                stage_v[r, sl] = stage_v[r, sl] * inv
            return carry2

        lax.fori_loop(0, WCH, div_row, 0)
        pltpu.sync_copy(stage_v, out_h.at[pl.ds(c * NPAD + r0, WCH)])
        return carry

    lax.fori_loop(0, STRIPE // WCH, wchunk, 0)


_SC_CACHE = {}


def _sc_mesh():
    return plsc.VectorSubcoreMesh(core_axis_name="c", subcore_axis_name="s")


def _cnt_sc(dst):
    if "cnt" not in _SC_CACHE:
        _SC_CACHE["cnt"] = functools.partial(
            pl.kernel,
            mesh=_sc_mesh(),
            out_type=jax.ShapeDtypeStruct((2 * NPAD, DH), jnp.float32),
            scratch_types=[
                pltpu.VMEM((CH2,), jnp.int32),
                pltpu.VMEM((CH2, DH), jnp.float32),
                pltpu.VMEM((WCH, DH), jnp.float32),
                pltpu.VMEM_SHARED((NPAD, DH), jnp.float32),
            ],
        )(_cnt_body)
    return _SC_CACHE["cnt"](dst)


def _seg_mean_sc(table, srcs2, dst, inv):
    if "seg" not in _SC_CACHE:
        _SC_CACHE["seg"] = functools.partial(
            pl.kernel,
            mesh=_sc_mesh(),
            out_type=jax.ShapeDtypeStruct((2 * NPAD, DH), jnp.float32),
            scratch_types=[
                pltpu.VMEM((CH,), jnp.int32),
                pltpu.VMEM((CH,), jnp.int32),
                pltpu.VMEM((CH, DH), jnp.float32),
                pltpu.VMEM((CH,), jnp.int32),
                pltpu.VMEM((CH,), jnp.int32),
                pltpu.VMEM((CH, DH), jnp.float32),
                pltpu.VMEM((WCH, DH), jnp.float32),
                pltpu.VMEM((WCH, LANES), jnp.float32),
                pltpu.VMEM_SHARED((NPAD, DH), jnp.float32),
                pltpu.SemaphoreType.DMA,
                pltpu.SemaphoreType.DMA,
            ],
        )(_seg_body)
    return _SC_CACHE["seg"](table, srcs2, dst, inv)


def _layer1_tc(a0, a1, x_split, Wl, bl, Wr0, Wr1, W2l, rows_per_blk=1000):
    grid = (N // rows_per_blk,)

    def body(a0_r, a1_r, xs_r, Wl_r, bl_r, Wr0_r, Wr1_r, W2_r, h_ref, y2_ref):
        mean = jnp.concatenate([a0_r[...], a1_r[...]], axis=1)
        pre = (_dot(mean, Wl_r[...]) + bl_r[...]
               + _dot(xs_r[0], Wr0_r[...]) + _dot(xs_r[1], Wr1_r[...]))
        h = _lk(pre)
        h_ref[...] = h
        y2 = _dot(h, W2_r[...])
        y2_ref[0] = y2[:, :DH]
        y2_ref[1] = y2[:, DH:]

    R = rows_per_blk
    return pl.pallas_call(
        body,
        grid=grid,
        in_specs=[
            pl.BlockSpec((R, DH), lambda i: (i, 0)),
            pl.BlockSpec((R, DH), lambda i: (i, 0)),
            pl.BlockSpec((2, R, DH), lambda i: (0, i, 0)),
            pl.BlockSpec(Wl.shape, lambda i: (0, 0)),
            pl.BlockSpec(bl.shape, lambda i: (0, 0)),
            pl.BlockSpec(Wr0.shape, lambda i: (0, 0)),
            pl.BlockSpec(Wr1.shape, lambda i: (0, 0)),
            pl.BlockSpec(W2l.shape, lambda i: (0, 0)),
        ],
        out_specs=[
            pl.BlockSpec((R, HID), lambda i: (i, 0)),
            pl.BlockSpec((2, R, DH), lambda i: (0, i, 0)),
        ],
        out_shape=[
            jax.ShapeDtypeStruct((N, HID), jnp.float32),
            jax.ShapeDtypeStruct((2, N, DH), jnp.float32),
        ],
    )(a0, a1, x_split, Wl, bl, Wr0, Wr1, W2l)


def _layer2_tc(b0, b1, h, Wr, bl, rows_per_blk=1000):
    grid = (N // rows_per_blk,)

    def body(b0_r, b1_r, h_r, Wr_r, bl_r, out_ref):
        mean = jnp.concatenate([b0_r[...], b1_r[...]], axis=1)
        out_ref[...] = mean + bl_r[...] + _dot(h_r[...], Wr_r[...])

    R = rows_per_blk
    return pl.pallas_call(
        body,
        grid=grid,
        in_specs=[
            pl.BlockSpec((R, DH), lambda i: (i, 0)),
            pl.BlockSpec((R, DH), lambda i: (i, 0)),
            pl.BlockSpec((R, HID), lambda i: (i, 0)),
            pl.BlockSpec(Wr.shape, lambda i: (0, 0)),
            pl.BlockSpec(bl.shape, lambda i: (0, 0)),
        ],
        out_specs=pl.BlockSpec((R, D), lambda i: (i, 0)),
        out_shape=jax.ShapeDtypeStruct((N, D), jnp.float32),
    )(b0, b1, h, Wr, bl)


def _pad_edges(ei):
    pad = EPAD - E
    src = jnp.concatenate([ei[0].astype(jnp.int32),
                           jnp.zeros((pad,), jnp.int32)])
    dst = jnp.concatenate([ei[1].astype(jnp.int32),
                           jnp.full((pad,), N, jnp.int32)])
    srcs2 = jnp.concatenate([src, src + N])
    return srcs2, dst


def _halves(seg_out):
    return seg_out[:N], seg_out[NPAD:NPAD + N]


def kernel(m_emb_feat, m_sim_feat, m_ass_feat, d_sim_feat, d_ass_feat,
           ei_md, ei_dm,
           fu_m_emb_W, fu_m_emb_b, fu_m_sim_W, fu_m_sim_b,
           fu_m_ass_W, fu_m_ass_b, fu_d_sim_W, fu_d_sim_b,
           fu_d_ass_W, fu_d_ass_b,
           m_emb_W, m_emb_b, m_sim_W, m_sim_b, m_ass_W, m_ass_b,
           d_sim_W, d_sim_b, d_ass_W, d_ass_b,
           w_m_emb, w_m_sim, w_m_ass, w_d_sim, w_d_ass,
           l1_md_Wl, l1_md_bl, l1_md_Wr,
           l2_md_Wl, l2_md_bl, l2_md_Wr,
           l1_dm_Wl, l1_dm_bl, l1_dm_Wr,
           l2_dm_Wl, l2_dm_bl, l2_dm_Wr):
    pm = [m_emb_W * w_m_emb, m_sim_W * w_m_sim, m_ass_W * w_m_ass]
    bm = (m_emb_b * w_m_emb + m_sim_b * w_m_sim
          + m_ass_b * w_m_ass).reshape(1, D)
    pd = [d_sim_W * w_d_sim, d_ass_W * w_d_ass]
    bd = (d_sim_b * w_d_sim + d_ass_b * w_d_ass).reshape(1, D)

    x_m = _features_tc(
        [m_emb_feat, m_sim_feat, m_ass_feat],
        [fu_m_emb_W, fu_m_sim_W, fu_m_ass_W],
        [fu_m_emb_b.reshape(1, -1), fu_m_sim_b.reshape(1, -1),
         fu_m_ass_b.reshape(1, -1)],
        pm, bm)
    x_d = _features_tc(
        [d_sim_feat, d_ass_feat],
        [fu_d_sim_W, fu_d_ass_W],
        [fu_d_sim_b.reshape(1, -1), fu_d_ass_b.reshape(1, -1)],
        pd, bd)

    src_md, dst_md = _pad_edges(ei_md)
    src_dm, dst_dm = _pad_edges(ei_dm)

    craw_md = _cnt_sc(dst_md)
    craw_dm = _cnt_sc(dst_dm)

    def _inv(craw):
        cnt = craw[:NPAD, 0] + craw[NPAD:, 0]
        return jnp.broadcast_to(
            (1.0 / jnp.maximum(cnt, 1.0))[:, None], (NPAD, LANES))

    inv_md = _inv(craw_md)
    inv_dm = _inv(craw_dm)

    mean_md = _seg_mean_sc(x_m.reshape(2 * N, DH), src_md, dst_md, inv_md)
    mean_dm = _seg_mean_sc(x_d.reshape(2 * N, DH), src_dm, dst_dm, inv_dm)

    a0_md, a1_md = _halves(mean_md)
    a0_dm, a1_dm = _halves(mean_dm)

    h_d, y2_dm = _layer1_tc(a0_md, a1_md, x_d,
                            l1_md_Wl, l1_md_bl.reshape(1, -1),
                            l1_md_Wr[:DH], l1_md_Wr[DH:], l2_dm_Wl)
    h_m, y2_md = _layer1_tc(a0_dm, a1_dm, x_m,
                            l1_dm_Wl, l1_dm_bl.reshape(1, -1),
                            l1_dm_Wr[:DH], l1_dm_Wr[DH:], l2_md_Wl)

    mean2_md = _seg_mean_sc(y2_md.reshape(2 * N, DH), src_md, dst_md, inv_md)
    mean2_dm = _seg_mean_sc(y2_dm.reshape(2 * N, DH), src_dm, dst_dm, inv_dm)

    b0_md, b1_md = _halves(mean2_md)
    b0_dm, b1_dm = _halves(mean2_dm)

    out_d = _layer2_tc(b0_md, b1_md, h_d, l2_md_Wr, l2_md_bl.reshape(1, -1))
    out_m = _layer2_tc(b0_dm, b1_dm, h_m, l2_dm_Wr, l2_dm_bl.reshape(1, -1))
    return jnp.concatenate([out_m, out_d], axis=0)


# speedup vs baseline: 3.8961x; 1.0008x over previous
import functools

import jax
import jax.numpy as jnp
from jax import lax
from jax.experimental import pallas as pl
from jax.experimental.pallas import tpu as pltpu
from jax.experimental.pallas import tpu_sc as plsc

N = 10000
E = 160000
D = 256
DH = 128
HID = 512
SLOPE = 0.2

LANES = 16
NSUB = 16
CH = 128
EPW = 10112
EPAD = NSUB * EPW
NPAD = 10240
STRIPE = NPAD // NSUB
CH2 = 64
EPW2 = EPAD // 32
WCH = 32


def _lk(x):
    return jnp.where(x >= 0, x, SLOPE * x)


def _dot(a, b):
    return jnp.dot(a, b, preferred_element_type=jnp.float32)


def _features_tc(feats, fu_ws, fu_bs, proj_ws, bsum, rows_per_blk=1000):
    nf = len(feats)
    grid = (N // rows_per_blk,)

    def body(*refs):
        frefs = refs[:nf]
        fw = refs[nf:2 * nf]
        fb = refs[2 * nf:3 * nf]
        pw = refs[3 * nf:4 * nf]
        bsum_r = refs[4 * nf]
        out_ref = refs[4 * nf + 1]
        acc = None
        for i in range(nf):
            u = _lk(_dot(frefs[i][...], fw[i][...]) + fb[i][...])
            t = _dot(u, pw[i][...])
            acc = t if acc is None else acc + t
        x = acc + bsum_r[...]
        out_ref[0] = x[:, :DH]
        out_ref[1] = x[:, DH:]

    in_specs = []
    for f in feats:
        d = f.shape[1]
        in_specs.append(pl.BlockSpec((rows_per_blk, d), lambda i: (i, 0)))
    for w in fu_ws:
        in_specs.append(pl.BlockSpec(w.shape, lambda i: (0, 0)))
    for b in fu_bs:
        in_specs.append(pl.BlockSpec(b.shape, lambda i: (0, 0)))
    for w in proj_ws:
        in_specs.append(pl.BlockSpec(w.shape, lambda i: (0, 0)))
    in_specs.append(pl.BlockSpec(bsum.shape, lambda i: (0, 0)))

    return pl.pallas_call(
        body,
        grid=grid,
        in_specs=in_specs,
        out_specs=pl.BlockSpec((2, rows_per_blk, DH), lambda i: (0, i, 0)),
        out_shape=jax.ShapeDtypeStruct((2, N, DH), jnp.float32),
    )(*feats, *fu_ws, *fu_bs, *proj_ws, bsum)


def _cnt_body(dst_h, out_h, dst_v, ones_v, cstage_v, cnt_sh):
    c = lax.axis_index("c")
    s = lax.axis_index("s")
    zero16 = jnp.zeros((LANES,), jnp.float32)
    one16 = jnp.ones((LANES,), jnp.float32)

    def zrow(r, carry):
        for j in range(DH // LANES):
            cstage_v[r, pl.ds(j * LANES, LANES)] = zero16
        return carry

    lax.fori_loop(0, WCH, zrow, 0)

    row0 = s * STRIPE

    def zchunk(k, carry):
        pltpu.sync_copy(cstage_v, cnt_sh.at[pl.ds(row0 + k * WCH, WCH)])
        return carry

    lax.fori_loop(0, STRIPE // WCH, zchunk, 0)

    def orow(r, carry):
        for j in range(DH // LANES):
            ones_v[r, pl.ds(j * LANES, LANES)] = one16
        return carry

    lax.fori_loop(0, CH2, orow, 0)
    plsc.subcore_barrier()

    ebase = c * (EPAD // 2) + s * EPW2

    def chunk(i, carry):
        pltpu.sync_copy(dst_h.at[pl.ds(ebase + i * CH2, CH2)], dst_v)
        pltpu.sync_copy(ones_v, cnt_sh.at[dst_v], add=True)
        return carry

    lax.fori_loop(0, EPW2 // CH2, chunk, 0)
    plsc.subcore_barrier()

    def wchunk(k, carry):
        r0 = row0 + k * WCH
        pltpu.sync_copy(cnt_sh.at[pl.ds(r0, WCH)], cstage_v)
        pltpu.sync_copy(cstage_v, out_h.at[pl.ds(c * NPAD + r0, WCH)])
        return carry

    lax.fori_loop(0, STRIPE // WCH, wchunk, 0)


def _seg_body(table_h, src_h, dst_h, inv_h, out_h,
              src0_v, dst0_v, rows0_v, src1_v, dst1_v, rows1_v,
              stage_v, cstage_v, acc_sh, semg0, sems0, semg1, sems1):
    c = lax.axis_index("c")
    s = lax.axis_index("s")
    zero16 = jnp.zeros((LANES,), jnp.float32)

    def zrow(r, carry):
        for j in range(DH // LANES):
            stage_v[r, pl.ds(j * LANES, LANES)] = zero16
        return carry

    lax.fori_loop(0, WCH, zrow, 0)

    row0 = s * STRIPE

    def zchunk(k, carry):
        pltpu.sync_copy(stage_v, acc_sh.at[pl.ds(row0 + k * WCH, WCH)])
        return carry

    lax.fori_loop(0, STRIPE // WCH, zchunk, 0)
    plsc.subcore_barrier()

    ebase = s * EPW
    sbase = c * EPAD + ebase
    bufs = ((src0_v, dst0_v, rows0_v, semg0, sems0),
            (src1_v, dst1_v, rows1_v, semg1, sems1))

    def load_idx(j, buf):
        pltpu.sync_copy(src_h.at[pl.ds(sbase + j * CH, CH)], buf[0])
        pltpu.sync_copy(dst_h.at[pl.ds(ebase + j * CH, CH)], buf[1])

    def wait_scat(buf):
        pltpu.make_async_copy(buf[2], acc_sh.at[buf[1]], buf[4]).wait()

    load_idx(0, bufs[0])
    pltpu.async_copy(table_h.at[src0_v], rows0_v, semg0)

    def pair(i2, carry):
        for p in range(2):
            j = i2 * 2 + p
            cur, nxt = bufs[p], bufs[1 - p]
            if p == 0:
                @pl.when(i2 > 0)
                def _():
                    wait_scat(nxt)
            else:
                wait_scat(nxt)
            load_idx(j + 1, nxt)
            pltpu.async_copy(table_h.at[nxt[0]], nxt[2], nxt[3])
            pltpu.make_async_copy(table_h.at[cur[0]], cur[2], cur[3]).wait()
            pltpu.async_copy(cur[2], acc_sh.at[cur[1]], cur[4], add=True)
        return carry

    lax.fori_loop(0, (EPW // CH) // 2, pair, 0)
    pltpu.make_async_copy(table_h.at[src0_v], rows0_v, semg0).wait()
    pltpu.async_copy(rows0_v, acc_sh.at[dst0_v], sems0, add=True)
    wait_scat(bufs[0])
    wait_scat(bufs[1])
    plsc.subcore_barrier()

    def wchunk(k, carry):
        r0 = row0 + k * WCH
        pltpu.sync_copy(acc_sh.at[pl.ds(r0, WCH)], stage_v)
        pltpu.sync_copy(inv_h.at[pl.ds(r0, WCH)], cstage_v)

        def div_row(r, carry2):
            inv = cstage_v[r, :]
            for j in range(DH // LANES):
                sl = pl.ds(j * LANES, LANES)
                stage_v[r, sl] = stage_v[r, sl] * inv
            return carry2

        lax.fori_loop(0, WCH, div_row, 0)
        pltpu.sync_copy(stage_v, out_h.at[pl.ds(c * NPAD + r0, WCH)])
        return carry

    lax.fori_loop(0, STRIPE // WCH, wchunk, 0)


_SC_CACHE = {}


def _sc_mesh():
    return plsc.VectorSubcoreMesh(core_axis_name="c", subcore_axis_name="s")


def _cnt_sc(dst):
    if "cnt" not in _SC_CACHE:
        _SC_CACHE["cnt"] = functools.partial(
            pl.kernel,
            mesh=_sc_mesh(),
            out_type=jax.ShapeDtypeStruct((2 * NPAD, DH), jnp.float32),
            scratch_types=[
                pltpu.VMEM((CH2,), jnp.int32),
                pltpu.VMEM((CH2, DH), jnp.float32),
                pltpu.VMEM((WCH, DH), jnp.float32),
                pltpu.VMEM_SHARED((NPAD, DH), jnp.float32),
            ],
        )(_cnt_body)
    return _SC_CACHE["cnt"](dst)


def _seg_mean_sc(table, srcs2, dst, inv):
    if "seg" not in _SC_CACHE:
        _SC_CACHE["seg"] = functools.partial(
            pl.kernel,
            mesh=_sc_mesh(),
            out_type=jax.ShapeDtypeStruct((2 * NPAD, DH), jnp.float32),
            scratch_types=[
                pltpu.VMEM((CH,), jnp.int32),
                pltpu.VMEM((CH,), jnp.int32),
                pltpu.VMEM((CH, DH), jnp.float32),
                pltpu.VMEM((CH,), jnp.int32),
                pltpu.VMEM((CH,), jnp.int32),
                pltpu.VMEM((CH, DH), jnp.float32),
                pltpu.VMEM((WCH, DH), jnp.float32),
                pltpu.VMEM((WCH, LANES), jnp.float32),
                pltpu.VMEM_SHARED((NPAD, DH), jnp.float32),
                pltpu.SemaphoreType.DMA,
                pltpu.SemaphoreType.DMA,
                pltpu.SemaphoreType.DMA,
                pltpu.SemaphoreType.DMA,
            ],
        )(_seg_body)
    return _SC_CACHE["seg"](table, srcs2, dst, inv)


def _layer1_tc(a0, a1, x_split, Wl, bl, Wr0, Wr1, W2l, rows_per_blk=1000):
    grid = (N // rows_per_blk,)

    def body(a0_r, a1_r, xs_r, Wl_r, bl_r, Wr0_r, Wr1_r, W2_r, h_ref, y2_ref):
        mean = jnp.concatenate([a0_r[...], a1_r[...]], axis=1)
        pre = (_dot(mean, Wl_r[...]) + bl_r[...]
               + _dot(xs_r[0], Wr0_r[...]) + _dot(xs_r[1], Wr1_r[...]))
        h = _lk(pre)
        h_ref[...] = h
        y2 = _dot(h, W2_r[...])
        y2_ref[0] = y2[:, :DH]
        y2_ref[1] = y2[:, DH:]

    R = rows_per_blk
    return pl.pallas_call(
        body,
        grid=grid,
        in_specs=[
            pl.BlockSpec((R, DH), lambda i: (i, 0)),
            pl.BlockSpec((R, DH), lambda i: (i, 0)),
            pl.BlockSpec((2, R, DH), lambda i: (0, i, 0)),
            pl.BlockSpec(Wl.shape, lambda i: (0, 0)),
            pl.BlockSpec(bl.shape, lambda i: (0, 0)),
            pl.BlockSpec(Wr0.shape, lambda i: (0, 0)),
            pl.BlockSpec(Wr1.shape, lambda i: (0, 0)),
            pl.BlockSpec(W2l.shape, lambda i: (0, 0)),
        ],
        out_specs=[
            pl.BlockSpec((R, HID), lambda i: (i, 0)),
            pl.BlockSpec((2, R, DH), lambda i: (0, i, 0)),
        ],
        out_shape=[
            jax.ShapeDtypeStruct((N, HID), jnp.float32),
            jax.ShapeDtypeStruct((2, N, DH), jnp.float32),
        ],
    )(a0, a1, x_split, Wl, bl, Wr0, Wr1, W2l)


def _layer2_tc(b0, b1, h, Wr, bl, rows_per_blk=1000):
    grid = (N // rows_per_blk,)

    def body(b0_r, b1_r, h_r, Wr_r, bl_r, out_ref):
        mean = jnp.concatenate([b0_r[...], b1_r[...]], axis=1)
        out_ref[...] = mean + bl_r[...] + _dot(h_r[...], Wr_r[...])

    R = rows_per_blk
    return pl.pallas_call(
        body,
        grid=grid,
        in_specs=[
            pl.BlockSpec((R, DH), lambda i: (i, 0)),
            pl.BlockSpec((R, DH), lambda i: (i, 0)),
            pl.BlockSpec((R, HID), lambda i: (i, 0)),
            pl.BlockSpec(Wr.shape, lambda i: (0, 0)),
            pl.BlockSpec(bl.shape, lambda i: (0, 0)),
        ],
        out_specs=pl.BlockSpec((R, D), lambda i: (i, 0)),
        out_shape=jax.ShapeDtypeStruct((N, D), jnp.float32),
    )(b0, b1, h, Wr, bl)


def _pad_edges(ei):
    pad = EPAD - E
    src = jnp.concatenate([ei[0].astype(jnp.int32),
                           jnp.zeros((pad,), jnp.int32)])
    dst = jnp.concatenate([ei[1].astype(jnp.int32),
                           jnp.full((pad,), N, jnp.int32)])
    srcs2 = jnp.concatenate([src, src + N])
    return srcs2, dst


def _halves(seg_out):
    return seg_out[:N], seg_out[NPAD:NPAD + N]


def kernel(m_emb_feat, m_sim_feat, m_ass_feat, d_sim_feat, d_ass_feat,
           ei_md, ei_dm,
           fu_m_emb_W, fu_m_emb_b, fu_m_sim_W, fu_m_sim_b,
           fu_m_ass_W, fu_m_ass_b, fu_d_sim_W, fu_d_sim_b,
           fu_d_ass_W, fu_d_ass_b,
           m_emb_W, m_emb_b, m_sim_W, m_sim_b, m_ass_W, m_ass_b,
           d_sim_W, d_sim_b, d_ass_W, d_ass_b,
           w_m_emb, w_m_sim, w_m_ass, w_d_sim, w_d_ass,
           l1_md_Wl, l1_md_bl, l1_md_Wr,
           l2_md_Wl, l2_md_bl, l2_md_Wr,
           l1_dm_Wl, l1_dm_bl, l1_dm_Wr,
           l2_dm_Wl, l2_dm_bl, l2_dm_Wr):
    pm = [m_emb_W * w_m_emb, m_sim_W * w_m_sim, m_ass_W * w_m_ass]
    bm = (m_emb_b * w_m_emb + m_sim_b * w_m_sim
          + m_ass_b * w_m_ass).reshape(1, D)
    pd = [d_sim_W * w_d_sim, d_ass_W * w_d_ass]
    bd = (d_sim_b * w_d_sim + d_ass_b * w_d_ass).reshape(1, D)

    x_m = _features_tc(
        [m_emb_feat, m_sim_feat, m_ass_feat],
        [fu_m_emb_W, fu_m_sim_W, fu_m_ass_W],
        [fu_m_emb_b.reshape(1, -1), fu_m_sim_b.reshape(1, -1),
         fu_m_ass_b.reshape(1, -1)],
        pm, bm)
    x_d = _features_tc(
        [d_sim_feat, d_ass_feat],
        [fu_d_sim_W, fu_d_ass_W],
        [fu_d_sim_b.reshape(1, -1), fu_d_ass_b.reshape(1, -1)],
        pd, bd)

    src_md, dst_md = _pad_edges(ei_md)
    src_dm, dst_dm = _pad_edges(ei_dm)

    craw_md = _cnt_sc(dst_md)
    craw_dm = _cnt_sc(dst_dm)

    def _inv(craw):
        cnt = craw[:NPAD, 0] + craw[NPAD:, 0]
        return jnp.broadcast_to(
            (1.0 / jnp.maximum(cnt, 1.0))[:, None], (NPAD, LANES))

    inv_md = _inv(craw_md)
    inv_dm = _inv(craw_dm)

    mean_md = _seg_mean_sc(x_m.reshape(2 * N, DH), src_md, dst_md, inv_md)
    mean_dm = _seg_mean_sc(x_d.reshape(2 * N, DH), src_dm, dst_dm, inv_dm)

    a0_md, a1_md = _halves(mean_md)
    a0_dm, a1_dm = _halves(mean_dm)

    h_d, y2_dm = _layer1_tc(a0_md, a1_md, x_d,
                            l1_md_Wl, l1_md_bl.reshape(1, -1),
                            l1_md_Wr[:DH], l1_md_Wr[DH:], l2_dm_Wl)
    h_m, y2_md = _layer1_tc(a0_dm, a1_dm, x_m,
                            l1_dm_Wl, l1_dm_bl.reshape(1, -1),
                            l1_dm_Wr[:DH], l1_dm_Wr[DH:], l2_md_Wl)

    mean2_md = _seg_mean_sc(y2_md.reshape(2 * N, DH), src_md, dst_md, inv_md)
    mean2_dm = _seg_mean_sc(y2_dm.reshape(2 * N, DH), src_dm, dst_dm, inv_dm)

    b0_md, b1_md = _halves(mean2_md)
    b0_dm, b1_dm = _halves(mean2_dm)

    out_d = _layer2_tc(b0_md, b1_md, h_d, l2_md_Wr, l2_md_bl.reshape(1, -1))
    out_m = _layer2_tc(b0_dm, b1_dm, h_m, l2_dm_Wr, l2_dm_bl.reshape(1, -1))
    return jnp.concatenate([out_m, out_d], axis=0)


# speedup vs baseline: 3.9437x; 1.0122x over previous
import functools

import jax
import jax.numpy as jnp
from jax import lax
from jax.experimental import pallas as pl
from jax.experimental.pallas import tpu as pltpu
from jax.experimental.pallas import tpu_sc as plsc

N = 10000
E = 160000
D = 256
DH = 128
HID = 512
SLOPE = 0.2

LANES = 16
NSUB = 16
CH = 128
EPW = 10112
EPAD = NSUB * EPW
NPAD = 10240
STRIPE = NPAD // NSUB
WCH = 32


def _lk(x):
    return jnp.where(x >= 0, x, SLOPE * x)


def _dot(a, b):
    return jnp.dot(a, b, preferred_element_type=jnp.float32)


def _features_tc(feats, fu_ws, fu_bs, proj_ws, bsum, rows_per_blk=1000):
    nf = len(feats)
    grid = (N // rows_per_blk,)

    def body(*refs):
        frefs = refs[:nf]
        fw = refs[nf:2 * nf]
        fb = refs[2 * nf:3 * nf]
        pw = refs[3 * nf:4 * nf]
        bsum_r = refs[4 * nf]
        out_ref = refs[4 * nf + 1]
        acc = None
        for i in range(nf):
            u = _lk(_dot(frefs[i][...], fw[i][...]) + fb[i][...])
            t = _dot(u, pw[i][...])
            acc = t if acc is None else acc + t
        x = acc + bsum_r[...]
        out_ref[0] = x[:, :DH]
        out_ref[1] = x[:, DH:]

    in_specs = []
    for f in feats:
        d = f.shape[1]
        in_specs.append(pl.BlockSpec((rows_per_blk, d), lambda i: (i, 0)))
    for w in fu_ws:
        in_specs.append(pl.BlockSpec(w.shape, lambda i: (0, 0)))
    for b in fu_bs:
        in_specs.append(pl.BlockSpec(b.shape, lambda i: (0, 0)))
    for w in proj_ws:
        in_specs.append(pl.BlockSpec(w.shape, lambda i: (0, 0)))
    in_specs.append(pl.BlockSpec(bsum.shape, lambda i: (0, 0)))

    return pl.pallas_call(
        body,
        grid=grid,
        in_specs=in_specs,
        out_specs=pl.BlockSpec((2, rows_per_blk, DH), lambda i: (0, i, 0)),
        out_shape=jax.ShapeDtypeStruct((2, N, DH), jnp.float32),
    )(*feats, *fu_ws, *fu_bs, *proj_ws, bsum)


def _cnt_body(dst_h, out_h, dst0_v, dst1_v, ones_v, stage_v, cstage_v,
              cnt_sh, sem0, sem1):
    c = lax.axis_index("c")
    s = lax.axis_index("s")
    zero16 = jnp.zeros((LANES,), jnp.float32)
    one16 = jnp.ones((LANES,), jnp.float32)

    def zrow(r, carry):
        for j in range(DH // LANES):
            stage_v[r, pl.ds(j * LANES, LANES)] = zero16
        return carry

    lax.fori_loop(0, WCH, zrow, 0)

    row0 = s * STRIPE

    def zchunk(k, carry):
        pltpu.sync_copy(stage_v, cnt_sh.at[pl.ds(row0 + k * WCH, WCH)])
        return carry

    lax.fori_loop(0, STRIPE // WCH, zchunk, 0)

    def orow(r, carry):
        for j in range(DH // LANES):
            ones_v[r, pl.ds(j * LANES, LANES)] = one16
        return carry

    lax.fori_loop(0, CH, orow, 0)
    plsc.subcore_barrier()

    ebase = c * EPAD + s * EPW
    bufs = ((dst0_v, sem0), (dst1_v, sem1))

    def wait_scat(buf):
        pltpu.make_async_copy(ones_v, cnt_sh.at[buf[0]], buf[1]).wait()

    pltpu.sync_copy(dst_h.at[pl.ds(ebase, CH)], dst0_v)

    def pair(i2, carry):
        for p in range(2):
            j = i2 * 2 + p
            cur, nxt = bufs[p], bufs[1 - p]
            if p == 0:
                @pl.when(i2 > 0)
                def _():
                    wait_scat(nxt)
            else:
                wait_scat(nxt)
            pltpu.sync_copy(dst_h.at[pl.ds(ebase + (j + 1) * CH, CH)],
                            nxt[0])
            pltpu.async_copy(ones_v, cnt_sh.at[cur[0]], cur[1], add=True)
        return carry

    lax.fori_loop(0, (EPW // CH) // 2, pair, 0)
    pltpu.async_copy(ones_v, cnt_sh.at[dst0_v], sem0, add=True)
    wait_scat(bufs[0])
    wait_scat(bufs[1])
    plsc.subcore_barrier()

    def wchunk(k, carry):
        r0 = row0 + k * WCH
        pltpu.sync_copy(cnt_sh.at[pl.ds(r0, WCH)], stage_v)

        def irow(r, carry2):
            cstage_v[r, :] = 1.0 / jnp.maximum(stage_v[r, pl.ds(0, LANES)],
                                               1.0)
            return carry2

        lax.fori_loop(0, WCH, irow, 0)
        pltpu.sync_copy(cstage_v, out_h.at[pl.ds(c * NPAD + r0, WCH)])
        return carry

    lax.fori_loop(0, STRIPE // WCH, wchunk, 0)


def _seg_body(table_h, src_h, dst_h, inv_h, out_h,
              src0_v, dst0_v, rows0_v, src1_v, dst1_v, rows1_v,
              stage_v, cstage_v, acc_sh, semg0, sems0, semg1, sems1):
    c = lax.axis_index("c")
    s = lax.axis_index("s")
    zero16 = jnp.zeros((LANES,), jnp.float32)

    def zrow(r, carry):
        for j in range(DH // LANES):
            stage_v[r, pl.ds(j * LANES, LANES)] = zero16
        return carry

    lax.fori_loop(0, WCH, zrow, 0)

    row0 = s * STRIPE

    def zchunk(k, carry):
        pltpu.sync_copy(stage_v, acc_sh.at[pl.ds(row0 + k * WCH, WCH)])
        return carry

    lax.fori_loop(0, STRIPE // WCH, zchunk, 0)
    plsc.subcore_barrier()

    ebase = s * EPW
    sbase = c * EPAD + ebase
    bufs = ((src0_v, dst0_v, rows0_v, semg0, sems0),
            (src1_v, dst1_v, rows1_v, semg1, sems1))

    def load_idx(j, buf):
        pltpu.sync_copy(src_h.at[pl.ds(sbase + j * CH, CH)], buf[0])
        pltpu.sync_copy(dst_h.at[pl.ds(ebase + j * CH, CH)], buf[1])

    def wait_scat(buf):
        pltpu.make_async_copy(buf[2], acc_sh.at[buf[1]], buf[4]).wait()

    load_idx(0, bufs[0])
    pltpu.async_copy(table_h.at[src0_v], rows0_v, semg0)

    def pair(i2, carry):
        for p in range(2):
            j = i2 * 2 + p
            cur, nxt = bufs[p], bufs[1 - p]
            if p == 0:
                @pl.when(i2 > 0)
                def _():
                    wait_scat(nxt)
            else:
                wait_scat(nxt)
            load_idx(j + 1, nxt)
            pltpu.async_copy(table_h.at[nxt[0]], nxt[2], nxt[3])
            pltpu.make_async_copy(table_h.at[cur[0]], cur[2], cur[3]).wait()
            pltpu.async_copy(cur[2], acc_sh.at[cur[1]], cur[4], add=True)
        return carry

    lax.fori_loop(0, (EPW // CH) // 2, pair, 0)
    pltpu.make_async_copy(table_h.at[src0_v], rows0_v, semg0).wait()
    pltpu.async_copy(rows0_v, acc_sh.at[dst0_v], sems0, add=True)
    wait_scat(bufs[0])
    wait_scat(bufs[1])
    plsc.subcore_barrier()

    def wchunk(k, carry):
        r0 = row0 + k * WCH
        pltpu.sync_copy(acc_sh.at[pl.ds(r0, WCH)], stage_v)
        pltpu.sync_copy(inv_h.at[pl.ds(r0, WCH)], cstage_v)

        def div_row(r, carry2):
            inv = cstage_v[r, :]
            for j in range(DH // LANES):
                sl = pl.ds(j * LANES, LANES)
                stage_v[r, sl] = stage_v[r, sl] * inv
            return carry2

        lax.fori_loop(0, WCH, div_row, 0)
        pltpu.sync_copy(stage_v, out_h.at[pl.ds(c * NPAD + r0, WCH)])
        return carry

    lax.fori_loop(0, STRIPE // WCH, wchunk, 0)


_SC_CACHE = {}


def _sc_mesh():
    return plsc.VectorSubcoreMesh(core_axis_name="c", subcore_axis_name="s")


def _cnt_sc(dst):
    if "cnt" not in _SC_CACHE:
        _SC_CACHE["cnt"] = functools.partial(
            pl.kernel,
            mesh=_sc_mesh(),
            out_type=jax.ShapeDtypeStruct((2 * NPAD, LANES), jnp.float32),
            scratch_types=[
                pltpu.VMEM((CH,), jnp.int32),
                pltpu.VMEM((CH,), jnp.int32),
                pltpu.VMEM((CH, DH), jnp.float32),
                pltpu.VMEM((WCH, DH), jnp.float32),
                pltpu.VMEM((WCH, LANES), jnp.float32),
                pltpu.VMEM_SHARED((NPAD, DH), jnp.float32),
                pltpu.SemaphoreType.DMA,
                pltpu.SemaphoreType.DMA,
            ],
        )(_cnt_body)
    return _SC_CACHE["cnt"](dst)


def _seg_mean_sc(table, srcs2, dst, inv):
    if "seg" not in _SC_CACHE:
        _SC_CACHE["seg"] = functools.partial(
            pl.kernel,
            mesh=_sc_mesh(),
            out_type=jax.ShapeDtypeStruct((2 * NPAD, DH), jnp.float32),
            scratch_types=[
                pltpu.VMEM((CH,), jnp.int32),
                pltpu.VMEM((CH,), jnp.int32),
                pltpu.VMEM((CH, DH), jnp.float32),
                pltpu.VMEM((CH,), jnp.int32),
                pltpu.VMEM((CH,), jnp.int32),
                pltpu.VMEM((CH, DH), jnp.float32),
                pltpu.VMEM((WCH, DH), jnp.float32),
                pltpu.VMEM((WCH, LANES), jnp.float32),
                pltpu.VMEM_SHARED((NPAD, DH), jnp.float32),
                pltpu.SemaphoreType.DMA,
                pltpu.SemaphoreType.DMA,
                pltpu.SemaphoreType.DMA,
                pltpu.SemaphoreType.DMA,
            ],
        )(_seg_body)
    return _SC_CACHE["seg"](table, srcs2, dst, inv)


def _layer1_tc(a0, a1, x_split, Wl, bl, Wr0, Wr1, W2l, rows_per_blk=1000):
    grid = (N // rows_per_blk,)

    def body(a0_r, a1_r, xs_r, Wl_r, bl_r, Wr0_r, Wr1_r, W2_r, h_ref, y2_ref):
        mean = jnp.concatenate([a0_r[...], a1_r[...]], axis=1)
        pre = (_dot(mean, Wl_r[...]) + bl_r[...]
               + _dot(xs_r[0], Wr0_r[...]) + _dot(xs_r[1], Wr1_r[...]))
        h = _lk(pre)
        h_ref[...] = h
        y2 = _dot(h, W2_r[...])
        y2_ref[0] = y2[:, :DH]
        y2_ref[1] = y2[:, DH:]

    R = rows_per_blk
    return pl.pallas_call(
        body,
        grid=grid,
        in_specs=[
            pl.BlockSpec((R, DH), lambda i: (i, 0)),
            pl.BlockSpec((R, DH), lambda i: (i, 0)),
            pl.BlockSpec((2, R, DH), lambda i: (0, i, 0)),
            pl.BlockSpec(Wl.shape, lambda i: (0, 0)),
            pl.BlockSpec(bl.shape, lambda i: (0, 0)),
            pl.BlockSpec(Wr0.shape, lambda i: (0, 0)),
            pl.BlockSpec(Wr1.shape, lambda i: (0, 0)),
            pl.BlockSpec(W2l.shape, lambda i: (0, 0)),
        ],
        out_specs=[
            pl.BlockSpec((R, HID), lambda i: (i, 0)),
            pl.BlockSpec((2, R, DH), lambda i: (0, i, 0)),
        ],
        out_shape=[
            jax.ShapeDtypeStruct((N, HID), jnp.float32),
            jax.ShapeDtypeStruct((2, N, DH), jnp.float32),
        ],
    )(a0, a1, x_split, Wl, bl, Wr0, Wr1, W2l)


def _layer2_tc(b0, b1, h, Wr, bl, rows_per_blk=1000):
    grid = (N // rows_per_blk,)

    def body(b0_r, b1_r, h_r, Wr_r, bl_r, out_ref):
        mean = jnp.concatenate([b0_r[...], b1_r[...]], axis=1)
        out_ref[...] = mean + bl_r[...] + _dot(h_r[...], Wr_r[...])

    R = rows_per_blk
    return pl.pallas_call(
        body,
        grid=grid,
        in_specs=[
            pl.BlockSpec((R, DH), lambda i: (i, 0)),
            pl.BlockSpec((R, DH), lambda i: (i, 0)),
            pl.BlockSpec((R, HID), lambda i: (i, 0)),
            pl.BlockSpec(Wr.shape, lambda i: (0, 0)),
            pl.BlockSpec(bl.shape, lambda i: (0, 0)),
        ],
        out_specs=pl.BlockSpec((R, D), lambda i: (i, 0)),
        out_shape=jax.ShapeDtypeStruct((N, D), jnp.float32),
    )(b0, b1, h, Wr, bl)


def _pad_edges(ei):
    pad = EPAD - E
    src = jnp.concatenate([ei[0].astype(jnp.int32),
                           jnp.zeros((pad,), jnp.int32)])
    dst = jnp.concatenate([ei[1].astype(jnp.int32),
                           jnp.full((pad,), N, jnp.int32)])
    srcs2 = jnp.concatenate([src, src + N])
    return srcs2, dst


def _halves(seg_out):
    return seg_out[:N], seg_out[NPAD:NPAD + N]


def kernel(m_emb_feat, m_sim_feat, m_ass_feat, d_sim_feat, d_ass_feat,
           ei_md, ei_dm,
           fu_m_emb_W, fu_m_emb_b, fu_m_sim_W, fu_m_sim_b,
           fu_m_ass_W, fu_m_ass_b, fu_d_sim_W, fu_d_sim_b,
           fu_d_ass_W, fu_d_ass_b,
           m_emb_W, m_emb_b, m_sim_W, m_sim_b, m_ass_W, m_ass_b,
           d_sim_W, d_sim_b, d_ass_W, d_ass_b,
           w_m_emb, w_m_sim, w_m_ass, w_d_sim, w_d_ass,
           l1_md_Wl, l1_md_bl, l1_md_Wr,
           l2_md_Wl, l2_md_bl, l2_md_Wr,
           l1_dm_Wl, l1_dm_bl, l1_dm_Wr,
           l2_dm_Wl, l2_dm_bl, l2_dm_Wr):
    pm = [m_emb_W * w_m_emb, m_sim_W * w_m_sim, m_ass_W * w_m_ass]
    bm = (m_emb_b * w_m_emb + m_sim_b * w_m_sim
          + m_ass_b * w_m_ass).reshape(1, D)
    pd = [d_sim_W * w_d_sim, d_ass_W * w_d_ass]
    bd = (d_sim_b * w_d_sim + d_ass_b * w_d_ass).reshape(1, D)

    x_m = _features_tc(
        [m_emb_feat, m_sim_feat, m_ass_feat],
        [fu_m_emb_W, fu_m_sim_W, fu_m_ass_W],
        [fu_m_emb_b.reshape(1, -1), fu_m_sim_b.reshape(1, -1),
         fu_m_ass_b.reshape(1, -1)],
        pm, bm)
    x_d = _features_tc(
        [d_sim_feat, d_ass_feat],
        [fu_d_sim_W, fu_d_ass_W],
        [fu_d_sim_b.reshape(1, -1), fu_d_ass_b.reshape(1, -1)],
        pd, bd)

    src_md, dst_md = _pad_edges(ei_md)
    src_dm, dst_dm = _pad_edges(ei_dm)

    invs = _cnt_sc(jnp.concatenate([dst_md, dst_dm]))
    inv_md = invs[:NPAD]
    inv_dm = invs[NPAD:]

    mean_md = _seg_mean_sc(x_m.reshape(2 * N, DH), src_md, dst_md, inv_md)
    mean_dm = _seg_mean_sc(x_d.reshape(2 * N, DH), src_dm, dst_dm, inv_dm)

    a0_md, a1_md = _halves(mean_md)
    a0_dm, a1_dm = _halves(mean_dm)

    h_d, y2_dm = _layer1_tc(a0_md, a1_md, x_d,
                            l1_md_Wl, l1_md_bl.reshape(1, -1),
                            l1_md_Wr[:DH], l1_md_Wr[DH:], l2_dm_Wl)
    h_m, y2_md = _layer1_tc(a0_dm, a1_dm, x_m,
                            l1_dm_Wl, l1_dm_bl.reshape(1, -1),
                            l1_dm_Wr[:DH], l1_dm_Wr[DH:], l2_md_Wl)

    mean2_md = _seg_mean_sc(y2_md.reshape(2 * N, DH), src_md, dst_md, inv_md)
    mean2_dm = _seg_mean_sc(y2_dm.reshape(2 * N, DH), src_dm, dst_dm, inv_dm)

    b0_md, b1_md = _halves(mean2_md)
    b0_dm, b1_dm = _halves(mean2_dm)

    out_d = _layer2_tc(b0_md, b1_md, h_d, l2_md_Wr, l2_md_bl.reshape(1, -1))
    out_m = _layer2_tc(b0_dm, b1_dm, h_m, l2_dm_Wr, l2_dm_bl.reshape(1, -1))
    return jnp.concatenate([out_m, out_d], axis=0)


# speedup vs baseline: 3.9452x; 1.0004x over previous
import functools

import jax
import jax.numpy as jnp
from jax import lax
from jax.experimental import pallas as pl
from jax.experimental.pallas import tpu as pltpu
from jax.experimental.pallas import tpu_sc as plsc

N = 10000
E = 160000
D = 256
DH = 128
HID = 512
SLOPE = 0.2

LANES = 16
NSUB = 16
CH = 128
EPW = 10112
EPAD = NSUB * EPW
NPAD = 10240
STRIPE = NPAD // NSUB
WCH = 32


def _lk(x):
    return jnp.where(x >= 0, x, SLOPE * x)


def _dot(a, b):
    return jnp.dot(a, b, preferred_element_type=jnp.float32)


def _features_tc(feats, fu_ws, fu_bs, proj_ws, bsum, rows_per_blk=1000):
    nf = len(feats)
    grid = (N // rows_per_blk,)

    def body(*refs):
        frefs = refs[:nf]
        fw = refs[nf:2 * nf]
        fb = refs[2 * nf:3 * nf]
        pw = refs[3 * nf:4 * nf]
        bsum_r = refs[4 * nf]
        out_ref = refs[4 * nf + 1]
        acc = None
        for i in range(nf):
            u = _lk(_dot(frefs[i][...], fw[i][...]) + fb[i][...])
            t = _dot(u, pw[i][...])
            acc = t if acc is None else acc + t
        x = acc + bsum_r[...]
        out_ref[0] = x[:, :DH]
        out_ref[1] = x[:, DH:]

    in_specs = []
    for f in feats:
        d = f.shape[1]
        in_specs.append(pl.BlockSpec((rows_per_blk, d), lambda i: (i, 0)))
    for w in fu_ws:
        in_specs.append(pl.BlockSpec(w.shape, lambda i: (0, 0)))
    for b in fu_bs:
        in_specs.append(pl.BlockSpec(b.shape, lambda i: (0, 0)))
    for w in proj_ws:
        in_specs.append(pl.BlockSpec(w.shape, lambda i: (0, 0)))
    in_specs.append(pl.BlockSpec(bsum.shape, lambda i: (0, 0)))

    return pl.pallas_call(
        body,
        grid=grid,
        in_specs=in_specs,
        out_specs=pl.BlockSpec((2, rows_per_blk, DH), lambda i: (0, i, 0)),
        out_shape=jax.ShapeDtypeStruct((2, N, DH), jnp.float32),
    )(*feats, *fu_ws, *fu_bs, *proj_ws, bsum)


def _cnt_body(dst_h, out_h, dst0_v, dst1_v, ones_v, stage_v, cstage_v,
              cnt_sh, sem0, sem1):
    c = lax.axis_index("c")
    s = lax.axis_index("s")
    zero16 = jnp.zeros((LANES,), jnp.float32)
    one16 = jnp.ones((LANES,), jnp.float32)

    def zrow(r, carry):
        for j in range(DH // LANES):
            stage_v[r, pl.ds(j * LANES, LANES)] = zero16
        return carry

    lax.fori_loop(0, WCH, zrow, 0)

    row0 = s * STRIPE

    def zchunk(k, carry):
        pltpu.sync_copy(stage_v, cnt_sh.at[pl.ds(row0 + k * WCH, WCH)])
        return carry

    lax.fori_loop(0, STRIPE // WCH, zchunk, 0)

    def orow(r, carry):
        for j in range(DH // LANES):
            ones_v[r, pl.ds(j * LANES, LANES)] = one16
        return carry

    lax.fori_loop(0, CH, orow, 0)
    plsc.subcore_barrier()

    ebase = c * EPAD + s * EPW
    bufs = ((dst0_v, sem0), (dst1_v, sem1))

    def wait_scat(buf):
        pltpu.make_async_copy(ones_v, cnt_sh.at[buf[0]], buf[1]).wait()

    pltpu.sync_copy(dst_h.at[pl.ds(ebase, CH)], dst0_v)

    def pair(i2, carry):
        for p in range(2):
            j = i2 * 2 + p
            cur, nxt = bufs[p], bufs[1 - p]
            if p == 0:
                @pl.when(i2 > 0)
                def _():
                    wait_scat(nxt)
            else:
                wait_scat(nxt)
            pltpu.sync_copy(dst_h.at[pl.ds(ebase + (j + 1) * CH, CH)],
                            nxt[0])
            pltpu.async_copy(ones_v, cnt_sh.at[cur[0]], cur[1], add=True)
        return carry

    lax.fori_loop(0, (EPW // CH) // 2, pair, 0)
    pltpu.async_copy(ones_v, cnt_sh.at[dst0_v], sem0, add=True)
    wait_scat(bufs[0])
    wait_scat(bufs[1])
    plsc.subcore_barrier()

    def wchunk(k, carry):
        r0 = row0 + k * WCH
        pltpu.sync_copy(cnt_sh.at[pl.ds(r0, WCH)], stage_v)

        def irow(r, carry2):
            cstage_v[r, :] = 1.0 / jnp.maximum(stage_v[r, pl.ds(0, LANES)],
                                               1.0)
            return carry2

        lax.fori_loop(0, WCH, irow, 0)
        pltpu.sync_copy(cstage_v, out_h.at[pl.ds(c * NPAD + r0, WCH)])
        return carry

    lax.fori_loop(0, STRIPE // WCH, wchunk, 0)


def _seg_body(table_h, src_h, dst_h, inv_h, out_h,
              src0_v, dst0_v, rows0_v, src1_v, dst1_v, rows1_v,
              stage_v, cstage_v, acc_sh, semg0, sems0, semg1, sems1):
    c = lax.axis_index("c")
    s = lax.axis_index("s")
    zero16 = jnp.zeros((LANES,), jnp.float32)

    def zrow(r, carry):
        for j in range(DH // LANES):
            stage_v[r, pl.ds(j * LANES, LANES)] = zero16
        return carry

    lax.fori_loop(0, WCH, zrow, 0)

    row0 = s * STRIPE

    def zchunk(k, carry):
        pltpu.sync_copy(stage_v, acc_sh.at[pl.ds(row0 + k * WCH, WCH)])
        return carry

    lax.fori_loop(0, STRIPE // WCH, zchunk, 0)
    plsc.subcore_barrier()

    ebase = s * EPW
    sbase = c * EPAD + ebase
    bufs = ((src0_v, dst0_v, rows0_v, semg0, sems0),
            (src1_v, dst1_v, rows1_v, semg1, sems1))

    def load_idx(j, buf):
        pltpu.sync_copy(src_h.at[pl.ds(sbase + j * CH, CH)], buf[0])
        pltpu.sync_copy(dst_h.at[pl.ds(ebase + j * CH, CH)], buf[1])

    def wait_scat(buf):
        pltpu.make_async_copy(buf[2], acc_sh.at[buf[1]], buf[4]).wait()

    load_idx(0, bufs[0])
    pltpu.async_copy(table_h.at[src0_v], rows0_v, semg0)

    def pair(i2, carry):
        for p in range(2):
            j = i2 * 2 + p
            cur, nxt = bufs[p], bufs[1 - p]
            if p == 0:
                @pl.when(i2 > 0)
                def _():
                    wait_scat(nxt)
            else:
                wait_scat(nxt)
            load_idx(j + 1, nxt)
            pltpu.async_copy(table_h.at[nxt[0]], nxt[2], nxt[3])
            pltpu.make_async_copy(table_h.at[cur[0]], cur[2], cur[3]).wait()
            pltpu.async_copy(cur[2], acc_sh.at[cur[1]], cur[4], add=True)
        return carry

    lax.fori_loop(0, (EPW // CH) // 2, pair, 0)
    pltpu.make_async_copy(table_h.at[src0_v], rows0_v, semg0).wait()
    pltpu.async_copy(rows0_v, acc_sh.at[dst0_v], sems0, add=True)
    wait_scat(bufs[0])
    wait_scat(bufs[1])
    plsc.subcore_barrier()

    def wchunk(k, carry):
        r0 = row0 + k * WCH
        pltpu.sync_copy(acc_sh.at[pl.ds(r0, WCH)], stage_v)
        pltpu.sync_copy(inv_h.at[pl.ds(r0, WCH)], cstage_v)

        def div_row(r, carry2):
            inv = cstage_v[r, :]
            for j in range(DH // LANES):
                sl = pl.ds(j * LANES, LANES)
                stage_v[r, sl] = stage_v[r, sl] * inv
            return carry2

        lax.fori_loop(0, WCH, div_row, 0)
        pltpu.sync_copy(stage_v, out_h.at[pl.ds(c * NPAD + r0, WCH)])
        return carry

    lax.fori_loop(0, STRIPE // WCH, wchunk, 0)


_SC_CACHE = {}


def _sc_mesh():
    return plsc.VectorSubcoreMesh(core_axis_name="c", subcore_axis_name="s")


def _cnt_sc(dst):
    if "cnt" not in _SC_CACHE:
        _SC_CACHE["cnt"] = functools.partial(
            pl.kernel,
            mesh=_sc_mesh(),
            out_type=jax.ShapeDtypeStruct((2 * NPAD, LANES), jnp.float32),
            scratch_types=[
                pltpu.VMEM((CH,), jnp.int32),
                pltpu.VMEM((CH,), jnp.int32),
                pltpu.VMEM((CH, DH), jnp.float32),
                pltpu.VMEM((WCH, DH), jnp.float32),
                pltpu.VMEM((WCH, LANES), jnp.float32),
                pltpu.VMEM_SHARED((NPAD, DH), jnp.float32),
                pltpu.SemaphoreType.DMA,
                pltpu.SemaphoreType.DMA,
            ],
        )(_cnt_body)
    return _SC_CACHE["cnt"](dst)


def _seg_mean_sc(table, srcs2, dst, inv):
    if "seg" not in _SC_CACHE:
        _SC_CACHE["seg"] = functools.partial(
            pl.kernel,
            mesh=_sc_mesh(),
            out_type=jax.ShapeDtypeStruct((2 * NPAD, DH), jnp.float32),
            scratch_types=[
                pltpu.VMEM((CH,), jnp.int32),
                pltpu.VMEM((CH,), jnp.int32),
                pltpu.VMEM((CH, DH), jnp.float32),
                pltpu.VMEM((CH,), jnp.int32),
                pltpu.VMEM((CH,), jnp.int32),
                pltpu.VMEM((CH, DH), jnp.float32),
                pltpu.VMEM((WCH, DH), jnp.float32),
                pltpu.VMEM((WCH, LANES), jnp.float32),
                pltpu.VMEM_SHARED((NPAD, DH), jnp.float32),
                pltpu.SemaphoreType.DMA,
                pltpu.SemaphoreType.DMA,
                pltpu.SemaphoreType.DMA,
                pltpu.SemaphoreType.DMA,
            ],
        )(_seg_body)
    return _SC_CACHE["seg"](table, srcs2, dst, inv)


def _layer1_tc(a0, a1, x_split, Wl, bl, Wr0, Wr1, W2l, rows_per_blk=1000):
    grid = (N // rows_per_blk,)

    def body(a0_r, a1_r, xs_r, Wl_r, bl_r, Wr0_r, Wr1_r, W2_r, h_ref, y2_ref):
        mean = jnp.concatenate([a0_r[...], a1_r[...]], axis=1)
        pre = (_dot(mean, Wl_r[...]) + bl_r[...]
               + _dot(xs_r[0], Wr0_r[...]) + _dot(xs_r[1], Wr1_r[...]))
        h = _lk(pre)
        h_ref[...] = h
        y2 = _dot(h, W2_r[...])
        y2_ref[0] = y2[:, :DH]
        y2_ref[1] = y2[:, DH:]

    R = rows_per_blk
    return pl.pallas_call(
        body,
        grid=grid,
        in_specs=[
            pl.BlockSpec((R, DH), lambda i: (i, 0)),
            pl.BlockSpec((R, DH), lambda i: (i, 0)),
            pl.BlockSpec((2, R, DH), lambda i: (0, i, 0)),
            pl.BlockSpec(Wl.shape, lambda i: (0, 0)),
            pl.BlockSpec(bl.shape, lambda i: (0, 0)),
            pl.BlockSpec(Wr0.shape, lambda i: (0, 0)),
            pl.BlockSpec(Wr1.shape, lambda i: (0, 0)),
            pl.BlockSpec(W2l.shape, lambda i: (0, 0)),
        ],
        out_specs=[
            pl.BlockSpec((R, HID), lambda i: (i, 0)),
            pl.BlockSpec((2, R, DH), lambda i: (0, i, 0)),
        ],
        out_shape=[
            jax.ShapeDtypeStruct((N, HID), jnp.float32),
            jax.ShapeDtypeStruct((2, N, DH), jnp.float32),
        ],
    )(a0, a1, x_split, Wl, bl, Wr0, Wr1, W2l)


def _layer2_tc(b0, b1, h, Wr, bl, rows_per_blk=1000):
    grid = (N // rows_per_blk,)

    def body(b0_r, b1_r, h_r, Wr_r, bl_r, out_ref):
        mean = jnp.concatenate([b0_r[...], b1_r[...]], axis=1)
        out_ref[...] = mean + bl_r[...] + _dot(h_r[...], Wr_r[...])

    R = rows_per_blk
    return pl.pallas_call(
        body,
        grid=grid,
        in_specs=[
            pl.BlockSpec((R, DH), lambda i: (i, 0)),
            pl.BlockSpec((R, DH), lambda i: (i, 0)),
            pl.BlockSpec((R, HID), lambda i: (i, 0)),
            pl.BlockSpec(Wr.shape, lambda i: (0, 0)),
            pl.BlockSpec(bl.shape, lambda i: (0, 0)),
        ],
        out_specs=pl.BlockSpec((R, D), lambda i: (i, 0)),
        out_shape=jax.ShapeDtypeStruct((N, D), jnp.float32),
    )(b0, b1, h, Wr, bl)


def _pad_edges(ei):
    pad = EPAD - E
    src = jnp.concatenate([ei[0].astype(jnp.int32),
                           jnp.zeros((pad,), jnp.int32)])
    dst = jnp.concatenate([ei[1].astype(jnp.int32),
                           jnp.full((pad,), N, jnp.int32)])
    srcs2 = jnp.concatenate([src, src + N])
    return srcs2, dst


def _halves(seg_out):
    return seg_out[:N], seg_out[NPAD:NPAD + N]


def kernel(m_emb_feat, m_sim_feat, m_ass_feat, d_sim_feat, d_ass_feat,
           ei_md, ei_dm,
           fu_m_emb_W, fu_m_emb_b, fu_m_sim_W, fu_m_sim_b,
           fu_m_ass_W, fu_m_ass_b, fu_d_sim_W, fu_d_sim_b,
           fu_d_ass_W, fu_d_ass_b,
           m_emb_W, m_emb_b, m_sim_W, m_sim_b, m_ass_W, m_ass_b,
           d_sim_W, d_sim_b, d_ass_W, d_ass_b,
           w_m_emb, w_m_sim, w_m_ass, w_d_sim, w_d_ass,
           l1_md_Wl, l1_md_bl, l1_md_Wr,
           l2_md_Wl, l2_md_bl, l2_md_Wr,
           l1_dm_Wl, l1_dm_bl, l1_dm_Wr,
           l2_dm_Wl, l2_dm_bl, l2_dm_Wr):
    pm = [m_emb_W * w_m_emb, m_sim_W * w_m_sim, m_ass_W * w_m_ass]
    bm = (m_emb_b * w_m_emb + m_sim_b * w_m_sim
          + m_ass_b * w_m_ass).reshape(1, D)
    pd = [d_sim_W * w_d_sim, d_ass_W * w_d_ass]
    bd = (d_sim_b * w_d_sim + d_ass_b * w_d_ass).reshape(1, D)

    src_md, dst_md = _pad_edges(ei_md)
    src_dm, dst_dm = _pad_edges(ei_dm)

    invs = _cnt_sc(jnp.concatenate([dst_md, dst_dm]))
    inv_md = invs[:NPAD]
    inv_dm = invs[NPAD:]

    x_m = _features_tc(
        [m_emb_feat, m_sim_feat, m_ass_feat],
        [fu_m_emb_W, fu_m_sim_W, fu_m_ass_W],
        [fu_m_emb_b.reshape(1, -1), fu_m_sim_b.reshape(1, -1),
         fu_m_ass_b.reshape(1, -1)],
        pm, bm)
    x_d = _features_tc(
        [d_sim_feat, d_ass_feat],
        [fu_d_sim_W, fu_d_ass_W],
        [fu_d_sim_b.reshape(1, -1), fu_d_ass_b.reshape(1, -1)],
        pd, bd)

    mean_md = _seg_mean_sc(x_m.reshape(2 * N, DH), src_md, dst_md, inv_md)
    mean_dm = _seg_mean_sc(x_d.reshape(2 * N, DH), src_dm, dst_dm, inv_dm)

    a0_md, a1_md = _halves(mean_md)
    a0_dm, a1_dm = _halves(mean_dm)

    h_d, y2_dm = _layer1_tc(a0_md, a1_md, x_d,
                            l1_md_Wl, l1_md_bl.reshape(1, -1),
                            l1_md_Wr[:DH], l1_md_Wr[DH:], l2_dm_Wl)
    h_m, y2_md = _layer1_tc(a0_dm, a1_dm, x_m,
                            l1_dm_Wl, l1_dm_bl.reshape(1, -1),
                            l1_dm_Wr[:DH], l1_dm_Wr[DH:], l2_md_Wl)

    mean2_md = _seg_mean_sc(y2_md.reshape(2 * N, DH), src_md, dst_md, inv_md)
    mean2_dm = _seg_mean_sc(y2_dm.reshape(2 * N, DH), src_dm, dst_dm, inv_dm)

    b0_md, b1_md = _halves(mean2_md)
    b0_dm, b1_dm = _halves(mean2_dm)

    out_d = _layer2_tc(b0_md, b1_md, h_d, l2_md_Wr, l2_md_bl.reshape(1, -1))
    out_m = _layer2_tc(b0_dm, b1_dm, h_m, l2_dm_Wr, l2_dm_bl.reshape(1, -1))
    return jnp.concatenate([out_m, out_d], axis=0)


# speedup vs baseline: 4.3228x; 1.0957x over previous
import functools

import jax
import jax.numpy as jnp
from jax import lax
from jax.experimental import pallas as pl
from jax.experimental.pallas import tpu as pltpu
from jax.experimental.pallas import tpu_sc as plsc

N = 10000
E = 160000
D = 256
DH = 128
HID = 512
SLOPE = 0.2

LANES = 16
NSUB = 16
CH = 128
EPW = 10112
EPAD = NSUB * EPW
NPAD = 10240
STRIPE = NPAD // NSUB
WCH = 32


def _lk(x):
    return jnp.where(x >= 0, x, SLOPE * x)


def _dot(a, b):
    return jnp.dot(a, b, preferred_element_type=jnp.float32)


def _features_tc(feats, fu_ws, fu_bs, proj_ws, bsum, rows_per_blk=1000):
    nf = len(feats)
    grid = (N // rows_per_blk,)

    def body(*refs):
        frefs = refs[:nf]
        fw = refs[nf:2 * nf]
        fb = refs[2 * nf:3 * nf]
        pw = refs[3 * nf:4 * nf]
        bsum_r = refs[4 * nf]
        out_ref = refs[4 * nf + 1]
        acc = None
        for i in range(nf):
            u = _lk(_dot(frefs[i][...], fw[i][...]) + fb[i][...])
            t = _dot(u, pw[i][...])
            acc = t if acc is None else acc + t
        x = acc + bsum_r[...]
        out_ref[0] = x[:, :DH]
        out_ref[1] = x[:, DH:]

    in_specs = []
    for f in feats:
        d = f.shape[1]
        in_specs.append(pl.BlockSpec((rows_per_blk, d), lambda i: (i, 0)))
    for w in fu_ws:
        in_specs.append(pl.BlockSpec(w.shape, lambda i: (0, 0)))
    for b in fu_bs:
        in_specs.append(pl.BlockSpec(b.shape, lambda i: (0, 0)))
    for w in proj_ws:
        in_specs.append(pl.BlockSpec(w.shape, lambda i: (0, 0)))
    in_specs.append(pl.BlockSpec(bsum.shape, lambda i: (0, 0)))

    return pl.pallas_call(
        body,
        grid=grid,
        in_specs=in_specs,
        out_specs=pl.BlockSpec((2, rows_per_blk, DH), lambda i: (0, i, 0)),
        out_shape=jax.ShapeDtypeStruct((2, N, DH), jnp.float32),
    )(*feats, *fu_ws, *fu_bs, *proj_ws, bsum)


def _cnt_body(dst_h, out_h, dst0_v, dst1_v, ones_v, stage_v, cstage_v,
              cnt_sh, sem0, sem1):
    c = lax.axis_index("c")
    s = lax.axis_index("s")
    zero16 = jnp.zeros((LANES,), jnp.float32)
    one16 = jnp.ones((LANES,), jnp.float32)

    def zrow(r, carry):
        for j in range(DH // LANES):
            stage_v[r, pl.ds(j * LANES, LANES)] = zero16
        return carry

    lax.fori_loop(0, WCH, zrow, 0)

    row0 = s * STRIPE

    def zchunk(k, carry):
        pltpu.sync_copy(stage_v, cnt_sh.at[pl.ds(row0 + k * WCH, WCH)])
        return carry

    lax.fori_loop(0, STRIPE // WCH, zchunk, 0)

    def orow(r, carry):
        for j in range(DH // LANES):
            ones_v[r, pl.ds(j * LANES, LANES)] = one16
        return carry

    lax.fori_loop(0, CH, orow, 0)
    plsc.subcore_barrier()

    ebase = c * EPAD + s * EPW
    bufs = ((dst0_v, sem0), (dst1_v, sem1))

    def wait_scat(buf):
        pltpu.make_async_copy(ones_v, cnt_sh.at[buf[0]], buf[1]).wait()

    pltpu.sync_copy(dst_h.at[pl.ds(ebase, CH)], dst0_v)

    def pair(i2, carry):
        for p in range(2):
            j = i2 * 2 + p
            cur, nxt = bufs[p], bufs[1 - p]
            if p == 0:
                @pl.when(i2 > 0)
                def _():
                    wait_scat(nxt)
            else:
                wait_scat(nxt)
            pltpu.sync_copy(dst_h.at[pl.ds(ebase + (j + 1) * CH, CH)],
                            nxt[0])
            pltpu.async_copy(ones_v, cnt_sh.at[cur[0]], cur[1], add=True)
        return carry

    lax.fori_loop(0, (EPW // CH) // 2, pair, 0)
    pltpu.async_copy(ones_v, cnt_sh.at[dst0_v], sem0, add=True)
    wait_scat(bufs[0])
    wait_scat(bufs[1])
    plsc.subcore_barrier()

    def wchunk(k, carry):
        r0 = row0 + k * WCH
        pltpu.sync_copy(cnt_sh.at[pl.ds(r0, WCH)], stage_v)

        def irow(r, carry2):
            cstage_v[r, :] = 1.0 / jnp.maximum(stage_v[r, pl.ds(0, LANES)],
                                               1.0)
            return carry2

        lax.fori_loop(0, WCH, irow, 0)
        pltpu.sync_copy(cstage_v, out_h.at[pl.ds(c * NPAD + r0, WCH)])
        return carry

    lax.fori_loop(0, STRIPE // WCH, wchunk, 0)


def _seg_body(table_h, packed_h, inv_h, out_h,
              big_v, src0_v, dst0_v, rows0_v, src1_v, dst1_v, rows1_v,
              stage_v, cstage_v, acc_sh, semg0, sems0, semg1, sems1):
    c = lax.axis_index("c")
    s = lax.axis_index("s")
    zero16 = jnp.zeros((LANES,), jnp.float32)

    def zrow(r, carry):
        for j in range(DH // LANES):
            stage_v[r, pl.ds(j * LANES, LANES)] = zero16
        return carry

    lax.fori_loop(0, WCH, zrow, 0)

    row0 = s * STRIPE

    def zchunk(k, carry):
        pltpu.sync_copy(stage_v, acc_sh.at[pl.ds(row0 + k * WCH, WCH)])
        return carry

    lax.fori_loop(0, STRIPE // WCH, zchunk, 0)
    plsc.subcore_barrier()

    nchunk = EPW // CH
    pbase = (c * NSUB + s) * (nchunk * 2 * CH)
    bufs = ((src0_v, dst0_v, rows0_v, semg0, sems0),
            (src1_v, dst1_v, rows1_v, semg1, sems1))

    def load_idx(j, buf):
        pltpu.sync_copy(packed_h.at[pl.ds(pbase + j * 2 * CH, 2 * CH)],
                        big_v)
        for q in range(CH // LANES):
            sl = pl.ds(q * LANES, LANES)
            buf[0][sl] = big_v[pl.ds(q * LANES, LANES)]
            buf[1][sl] = big_v[pl.ds(CH + q * LANES, LANES)]

    def wait_scat(buf):
        pltpu.make_async_copy(buf[2], acc_sh.at[buf[1]], buf[4]).wait()

    load_idx(0, bufs[0])
    pltpu.async_copy(table_h.at[src0_v], rows0_v, semg0)

    def pair(i2, carry):
        for p in range(2):
            j = i2 * 2 + p
            cur, nxt = bufs[p], bufs[1 - p]
            if p == 0:
                @pl.when(i2 > 0)
                def _():
                    wait_scat(nxt)
            else:
                wait_scat(nxt)
            load_idx(j + 1, nxt)
            pltpu.async_copy(table_h.at[nxt[0]], nxt[2], nxt[3])
            pltpu.make_async_copy(table_h.at[cur[0]], cur[2], cur[3]).wait()
            pltpu.async_copy(cur[2], acc_sh.at[cur[1]], cur[4], add=True)
        return carry

    lax.fori_loop(0, (EPW // CH) // 2, pair, 0)
    pltpu.make_async_copy(table_h.at[src0_v], rows0_v, semg0).wait()
    pltpu.async_copy(rows0_v, acc_sh.at[dst0_v], sems0, add=True)
    wait_scat(bufs[0])
    wait_scat(bufs[1])
    plsc.subcore_barrier()

    def wchunk(k, carry):
        r0 = row0 + k * WCH
        pltpu.sync_copy(acc_sh.at[pl.ds(r0, WCH)], stage_v)
        pltpu.sync_copy(inv_h.at[pl.ds(r0, WCH)], cstage_v)

        def div_row(r, carry2):
            inv = cstage_v[r, :]
            for j in range(DH // LANES):
                sl = pl.ds(j * LANES, LANES)
                stage_v[r, sl] = stage_v[r, sl] * inv
            return carry2

        lax.fori_loop(0, WCH, div_row, 0)
        pltpu.sync_copy(stage_v, out_h.at[pl.ds(c * NPAD + r0, WCH)])
        return carry

    lax.fori_loop(0, STRIPE // WCH, wchunk, 0)


_SC_CACHE = {}


def _sc_mesh():
    return plsc.VectorSubcoreMesh(core_axis_name="c", subcore_axis_name="s")


def _cnt_sc(dst):
    if "cnt" not in _SC_CACHE:
        _SC_CACHE["cnt"] = functools.partial(
            pl.kernel,
            mesh=_sc_mesh(),
            out_type=jax.ShapeDtypeStruct((2 * NPAD, LANES), jnp.float32),
            scratch_types=[
                pltpu.VMEM((CH,), jnp.int32),
                pltpu.VMEM((CH,), jnp.int32),
                pltpu.VMEM((CH, DH), jnp.float32),
                pltpu.VMEM((WCH, DH), jnp.float32),
                pltpu.VMEM((WCH, LANES), jnp.float32),
                pltpu.VMEM_SHARED((NPAD, DH), jnp.float32),
                pltpu.SemaphoreType.DMA,
                pltpu.SemaphoreType.DMA,
            ],
        )(_cnt_body)
    return _SC_CACHE["cnt"](dst)


def _seg_mean_sc(table, packed, inv):
    if "seg" not in _SC_CACHE:
        _SC_CACHE["seg"] = functools.partial(
            pl.kernel,
            mesh=_sc_mesh(),
            out_type=jax.ShapeDtypeStruct((2 * NPAD, DH), jnp.float32),
            scratch_types=[
                pltpu.VMEM((2 * CH,), jnp.int32),
                pltpu.VMEM((CH,), jnp.int32),
                pltpu.VMEM((CH,), jnp.int32),
                pltpu.VMEM((CH, DH), jnp.float32),
                pltpu.VMEM((CH,), jnp.int32),
                pltpu.VMEM((CH,), jnp.int32),
                pltpu.VMEM((CH, DH), jnp.float32),
                pltpu.VMEM((WCH, DH), jnp.float32),
                pltpu.VMEM((WCH, LANES), jnp.float32),
                pltpu.VMEM_SHARED((NPAD, DH), jnp.float32),
                pltpu.SemaphoreType.DMA,
                pltpu.SemaphoreType.DMA,
                pltpu.SemaphoreType.DMA,
                pltpu.SemaphoreType.DMA,
            ],
        )(_seg_body)
    return _SC_CACHE["seg"](table, packed, inv)


def _layer1_tc(a0, a1, x_split, Wl, bl, Wr0, Wr1, W2l, rows_per_blk=1000):
    grid = (N // rows_per_blk,)

    def body(a0_r, a1_r, xs_r, Wl_r, bl_r, Wr0_r, Wr1_r, W2_r, h_ref, y2_ref):
        mean = jnp.concatenate([a0_r[...], a1_r[...]], axis=1)
        pre = (_dot(mean, Wl_r[...]) + bl_r[...]
               + _dot(xs_r[0], Wr0_r[...]) + _dot(xs_r[1], Wr1_r[...]))
        h = _lk(pre)
        h_ref[...] = h
        y2 = _dot(h, W2_r[...])
        y2_ref[0] = y2[:, :DH]
        y2_ref[1] = y2[:, DH:]

    R = rows_per_blk
    return pl.pallas_call(
        body,
        grid=grid,
        in_specs=[
            pl.BlockSpec((R, DH), lambda i: (i, 0)),
            pl.BlockSpec((R, DH), lambda i: (i, 0)),
            pl.BlockSpec((2, R, DH), lambda i: (0, i, 0)),
            pl.BlockSpec(Wl.shape, lambda i: (0, 0)),
            pl.BlockSpec(bl.shape, lambda i: (0, 0)),
            pl.BlockSpec(Wr0.shape, lambda i: (0, 0)),
            pl.BlockSpec(Wr1.shape, lambda i: (0, 0)),
            pl.BlockSpec(W2l.shape, lambda i: (0, 0)),
        ],
        out_specs=[
            pl.BlockSpec((R, HID), lambda i: (i, 0)),
            pl.BlockSpec((2, R, DH), lambda i: (0, i, 0)),
        ],
        out_shape=[
            jax.ShapeDtypeStruct((N, HID), jnp.float32),
            jax.ShapeDtypeStruct((2, N, DH), jnp.float32),
        ],
    )(a0, a1, x_split, Wl, bl, Wr0, Wr1, W2l)


def _layer2_tc(b0, b1, h, Wr, bl, rows_per_blk=1000):
    grid = (N // rows_per_blk,)

    def body(b0_r, b1_r, h_r, Wr_r, bl_r, out_ref):
        mean = jnp.concatenate([b0_r[...], b1_r[...]], axis=1)
        out_ref[...] = mean + bl_r[...] + _dot(h_r[...], Wr_r[...])

    R = rows_per_blk
    return pl.pallas_call(
        body,
        grid=grid,
        in_specs=[
            pl.BlockSpec((R, DH), lambda i: (i, 0)),
            pl.BlockSpec((R, DH), lambda i: (i, 0)),
            pl.BlockSpec((R, HID), lambda i: (i, 0)),
            pl.BlockSpec(Wr.shape, lambda i: (0, 0)),
            pl.BlockSpec(bl.shape, lambda i: (0, 0)),
        ],
        out_specs=pl.BlockSpec((R, D), lambda i: (i, 0)),
        out_shape=jax.ShapeDtypeStruct((N, D), jnp.float32),
    )(b0, b1, h, Wr, bl)


def _pad_edges(ei):
    pad = EPAD - E
    src = jnp.concatenate([ei[0].astype(jnp.int32),
                           jnp.zeros((pad,), jnp.int32)])
    dst = jnp.concatenate([ei[1].astype(jnp.int32),
                           jnp.full((pad,), N, jnp.int32)])
    srcs2 = jnp.stack([src, src + N])
    nchunk = EPW // CH
    src_r = srcs2.reshape(2, NSUB, nchunk, CH)
    dst_r = jnp.broadcast_to(dst.reshape(1, NSUB, nchunk, CH),
                             (2, NSUB, nchunk, CH))
    packed = jnp.concatenate([src_r, dst_r], axis=3).reshape(-1)
    return packed, dst


def _halves(seg_out):
    return seg_out[:N], seg_out[NPAD:NPAD + N]


def kernel(m_emb_feat, m_sim_feat, m_ass_feat, d_sim_feat, d_ass_feat,
           ei_md, ei_dm,
           fu_m_emb_W, fu_m_emb_b, fu_m_sim_W, fu_m_sim_b,
           fu_m_ass_W, fu_m_ass_b, fu_d_sim_W, fu_d_sim_b,
           fu_d_ass_W, fu_d_ass_b,
           m_emb_W, m_emb_b, m_sim_W, m_sim_b, m_ass_W, m_ass_b,
           d_sim_W, d_sim_b, d_ass_W, d_ass_b,
           w_m_emb, w_m_sim, w_m_ass, w_d_sim, w_d_ass,
           l1_md_Wl, l1_md_bl, l1_md_Wr,
           l2_md_Wl, l2_md_bl, l2_md_Wr,
           l1_dm_Wl, l1_dm_bl, l1_dm_Wr,
           l2_dm_Wl, l2_dm_bl, l2_dm_Wr):
    pm = [m_emb_W * w_m_emb, m_sim_W * w_m_sim, m_ass_W * w_m_ass]
    bm = (m_emb_b * w_m_emb + m_sim_b * w_m_sim
          + m_ass_b * w_m_ass).reshape(1, D)
    pd = [d_sim_W * w_d_sim, d_ass_W * w_d_ass]
    bd = (d_sim_b * w_d_sim + d_ass_b * w_d_ass).reshape(1, D)

    src_md, dst_md = _pad_edges(ei_md)
    src_dm, dst_dm = _pad_edges(ei_dm)

    invs = _cnt_sc(jnp.concatenate([dst_md, dst_dm]))
    inv_md = invs[:NPAD]
    inv_dm = invs[NPAD:]

    x_m = _features_tc(
        [m_emb_feat, m_sim_feat, m_ass_feat],
        [fu_m_emb_W, fu_m_sim_W, fu_m_ass_W],
        [fu_m_emb_b.reshape(1, -1), fu_m_sim_b.reshape(1, -1),
         fu_m_ass_b.reshape(1, -1)],
        pm, bm)
    x_d = _features_tc(
        [d_sim_feat, d_ass_feat],
        [fu_d_sim_W, fu_d_ass_W],
        [fu_d_sim_b.reshape(1, -1), fu_d_ass_b.reshape(1, -1)],
        pd, bd)

    mean_md = _seg_mean_sc(x_m.reshape(2 * N, DH), src_md, inv_md)
    mean_dm = _seg_mean_sc(x_d.reshape(2 * N, DH), src_dm, inv_dm)

    a0_md, a1_md = _halves(mean_md)
    a0_dm, a1_dm = _halves(mean_dm)

    h_d, y2_dm = _layer1_tc(a0_md, a1_md, x_d,
                            l1_md_Wl, l1_md_bl.reshape(1, -1),
                            l1_md_Wr[:DH], l1_md_Wr[DH:], l2_dm_Wl)
    h_m, y2_md = _layer1_tc(a0_dm, a1_dm, x_m,
                            l1_dm_Wl, l1_dm_bl.reshape(1, -1),
                            l1_dm_Wr[:DH], l1_dm_Wr[DH:], l2_md_Wl)

    mean2_md = _seg_mean_sc(y2_md.reshape(2 * N, DH), src_md, inv_md)
    mean2_dm = _seg_mean_sc(y2_dm.reshape(2 * N, DH), src_dm, inv_dm)

    b0_md, b1_md = _halves(mean2_md)
    b0_dm, b1_dm = _halves(mean2_dm)

    out_d = _layer2_tc(b0_md, b1_md, h_d, l2_md_Wr, l2_md_bl.reshape(1, -1))
    out_m = _layer2_tc(b0_dm, b1_dm, h_m, l2_dm_Wr, l2_dm_bl.reshape(1, -1))
    return jnp.concatenate([out_m, out_d], axis=0)


# speedup vs baseline: 4.5257x; 1.0469x over previous
import functools

import jax
import jax.numpy as jnp
from jax import lax
from jax.experimental import pallas as pl
from jax.experimental.pallas import tpu as pltpu
from jax.experimental.pallas import tpu_sc as plsc

N = 10000
E = 160000
D = 256
DH = 128
HID = 512
SLOPE = 0.2

LANES = 16
NSUB = 16
CH = 128
EPW = 10112
EPAD = NSUB * EPW
NPAD = 10240
STRIPE = NPAD // NSUB
WCH = 32


def _lk(x):
    return jnp.where(x >= 0, x, SLOPE * x)


def _dot(a, b):
    return jnp.dot(a, b, preferred_element_type=jnp.float32)


def _features_tc(feats, fu_ws, fu_bs, proj_ws, bsum, rows_per_blk=1000):
    nf = len(feats)
    grid = (N // rows_per_blk,)

    def body(*refs):
        frefs = refs[:nf]
        fw = refs[nf:2 * nf]
        fb = refs[2 * nf:3 * nf]
        pw = refs[3 * nf:4 * nf]
        bsum_r = refs[4 * nf]
        out_ref = refs[4 * nf + 1]
        acc = None
        for i in range(nf):
            u = _lk(_dot(frefs[i][...], fw[i][...]) + fb[i][...])
            t = _dot(u, pw[i][...])
            acc = t if acc is None else acc + t
        x = acc + bsum_r[...]
        out_ref[0] = x[:, :DH]
        out_ref[1] = x[:, DH:]

    in_specs = []
    for f in feats:
        d = f.shape[1]
        in_specs.append(pl.BlockSpec((rows_per_blk, d), lambda i: (i, 0)))
    for w in fu_ws:
        in_specs.append(pl.BlockSpec(w.shape, lambda i: (0, 0)))
    for b in fu_bs:
        in_specs.append(pl.BlockSpec(b.shape, lambda i: (0, 0)))
    for w in proj_ws:
        in_specs.append(pl.BlockSpec(w.shape, lambda i: (0, 0)))
    in_specs.append(pl.BlockSpec(bsum.shape, lambda i: (0, 0)))

    return pl.pallas_call(
        body,
        grid=grid,
        in_specs=in_specs,
        out_specs=pl.BlockSpec((2, rows_per_blk, DH), lambda i: (0, i, 0)),
        out_shape=jax.ShapeDtypeStruct((2, N, DH), jnp.float32),
    )(*feats, *fu_ws, *fu_bs, *proj_ws, bsum)


def _cnt_body(dst_h, out_h, dst0_v, dst1_v, ones_v, stage_v, cstage_v,
              cnt_sh, sem0, sem1):
    c = lax.axis_index("c")
    s = lax.axis_index("s")
    zero16 = jnp.zeros((LANES,), jnp.float32)
    one16 = jnp.ones((LANES,), jnp.float32)

    def zrow(r, carry):
        for j in range(DH // LANES):
            stage_v[r, pl.ds(j * LANES, LANES)] = zero16
        return carry

    lax.fori_loop(0, WCH, zrow, 0)

    row0 = s * STRIPE

    def zchunk(k, carry):
        pltpu.sync_copy(stage_v, cnt_sh.at[pl.ds(row0 + k * WCH, WCH)])
        return carry

    lax.fori_loop(0, STRIPE // WCH, zchunk, 0)

    def orow(r, carry):
        for j in range(DH // LANES):
            ones_v[r, pl.ds(j * LANES, LANES)] = one16
        return carry

    lax.fori_loop(0, CH, orow, 0)
    plsc.subcore_barrier()

    ebase = c * EPAD + s * EPW
    bufs = ((dst0_v, sem0), (dst1_v, sem1))

    def wait_scat(buf):
        pltpu.make_async_copy(ones_v, cnt_sh.at[buf[0]], buf[1]).wait()

    pltpu.sync_copy(dst_h.at[pl.ds(ebase, CH)], dst0_v)

    def pair(i2, carry):
        for p in range(2):
            j = i2 * 2 + p
            cur, nxt = bufs[p], bufs[1 - p]
            if p == 0:
                @pl.when(i2 > 0)
                def _():
                    wait_scat(nxt)
            else:
                wait_scat(nxt)
            pltpu.sync_copy(dst_h.at[pl.ds(ebase + (j + 1) * CH, CH)],
                            nxt[0])
            pltpu.async_copy(ones_v, cnt_sh.at[cur[0]], cur[1], add=True)
        return carry

    lax.fori_loop(0, (EPW // CH) // 2, pair, 0)
    pltpu.async_copy(ones_v, cnt_sh.at[dst0_v], sem0, add=True)
    wait_scat(bufs[0])
    wait_scat(bufs[1])
    plsc.subcore_barrier()

    def wchunk(k, carry):
        r0 = row0 + k * WCH
        pltpu.sync_copy(cnt_sh.at[pl.ds(r0, WCH)], stage_v)

        def irow(r, carry2):
            cstage_v[r, :] = 1.0 / jnp.maximum(stage_v[r, pl.ds(0, LANES)],
                                               1.0)
            return carry2

        lax.fori_loop(0, WCH, irow, 0)
        pltpu.sync_copy(cstage_v, out_h.at[pl.ds(c * NPAD + r0, WCH)])
        return carry

    lax.fori_loop(0, STRIPE // WCH, wchunk, 0)


def _seg_body(table_h, packed_h, inv_h, out_h,
              big0_v, big1_v, src0_v, dst0_v, rows0_v,
              src1_v, dst1_v, rows1_v,
              stage_v, cstage_v, acc_sh,
              semg0, sems0, semg1, sems1, semi0, semi1):
    c = lax.axis_index("c")
    s = lax.axis_index("s")
    zero16 = jnp.zeros((LANES,), jnp.float32)

    def zrow(r, carry):
        for j in range(DH // LANES):
            stage_v[r, pl.ds(j * LANES, LANES)] = zero16
        return carry

    lax.fori_loop(0, WCH, zrow, 0)

    row0 = s * STRIPE

    def zchunk(k, carry):
        pltpu.sync_copy(stage_v, acc_sh.at[pl.ds(row0 + k * WCH, WCH)])
        return carry

    lax.fori_loop(0, STRIPE // WCH, zchunk, 0)
    plsc.subcore_barrier()

    nchunk = EPW // CH
    npair = nchunk // 2
    pbase = (c * NSUB + s) * (nchunk * 2 * CH)
    bufs = ((src0_v, dst0_v, rows0_v, semg0, sems0),
            (src1_v, dst1_v, rows1_v, semg1, sems1))
    bigs = ((big0_v, semi0), (big1_v, semi1))

    def pslice(j):
        return packed_h.at[pl.ds(pbase + j * 2 * CH, 2 * CH)]

    def unpack(big, buf):
        for q in range(CH // LANES):
            sl = pl.ds(q * LANES, LANES)
            buf[0][sl] = big[0][pl.ds(q * LANES, LANES)]
            buf[1][sl] = big[0][pl.ds(CH + q * LANES, LANES)]

    def wait_scat(buf):
        pltpu.make_async_copy(buf[2], acc_sh.at[buf[1]], buf[4]).wait()

    pltpu.async_copy(pslice(0), big0_v, semi0)
    pltpu.make_async_copy(pslice(0), big0_v, semi0).wait()
    unpack(bigs[0], bufs[0])
    pltpu.async_copy(pslice(1), big1_v, semi1)
    pltpu.async_copy(table_h.at[src0_v], rows0_v, semg0)

    def pair(i2, carry):
        for p in range(2):
            j = i2 * 2 + p
            cur, nxt = bufs[p], bufs[1 - p]
            bigc, bign = bigs[p], bigs[1 - p]
            if p == 0:
                @pl.when(i2 > 0)
                def _():
                    wait_scat(nxt)
            else:
                wait_scat(nxt)
            pltpu.make_async_copy(pslice(j + 1), bign[0], bign[1]).wait()
            unpack(bign, nxt)
            if p == 0:
                pltpu.async_copy(pslice(j + 2), bigc[0], bigc[1])
            else:
                @pl.when(i2 < npair - 1)
                def _():
                    pltpu.async_copy(pslice(j + 2), bigc[0], bigc[1])
            pltpu.async_copy(table_h.at[nxt[0]], nxt[2], nxt[3])
            pltpu.make_async_copy(table_h.at[cur[0]], cur[2], cur[3]).wait()
            pltpu.async_copy(cur[2], acc_sh.at[cur[1]], cur[4], add=True)
        return carry

    lax.fori_loop(0, npair, pair, 0)
    pltpu.make_async_copy(table_h.at[src0_v], rows0_v, semg0).wait()
    pltpu.async_copy(rows0_v, acc_sh.at[dst0_v], sems0, add=True)
    wait_scat(bufs[0])
    wait_scat(bufs[1])
    plsc.subcore_barrier()

    def wchunk(k, carry):
        r0 = row0 + k * WCH
        pltpu.sync_copy(acc_sh.at[pl.ds(r0, WCH)], stage_v)
        pltpu.sync_copy(inv_h.at[pl.ds(r0, WCH)], cstage_v)

        def div_row(r, carry2):
            inv = cstage_v[r, :]
            for j in range(DH // LANES):
                sl = pl.ds(j * LANES, LANES)
                stage_v[r, sl] = stage_v[r, sl] * inv
            return carry2

        lax.fori_loop(0, WCH, div_row, 0)
        pltpu.sync_copy(stage_v, out_h.at[pl.ds(c * NPAD + r0, WCH)])
        return carry

    lax.fori_loop(0, STRIPE // WCH, wchunk, 0)


_SC_CACHE = {}


def _sc_mesh():
    return plsc.VectorSubcoreMesh(core_axis_name="c", subcore_axis_name="s")


def _cnt_sc(dst):
    if "cnt" not in _SC_CACHE:
        _SC_CACHE["cnt"] = functools.partial(
            pl.kernel,
            mesh=_sc_mesh(),
            out_type=jax.ShapeDtypeStruct((2 * NPAD, LANES), jnp.float32),
            scratch_types=[
                pltpu.VMEM((CH,), jnp.int32),
                pltpu.VMEM((CH,), jnp.int32),
                pltpu.VMEM((CH, DH), jnp.float32),
                pltpu.VMEM((WCH, DH), jnp.float32),
                pltpu.VMEM((WCH, LANES), jnp.float32),
                pltpu.VMEM_SHARED((NPAD, DH), jnp.float32),
                pltpu.SemaphoreType.DMA,
                pltpu.SemaphoreType.DMA,
            ],
        )(_cnt_body)
    return _SC_CACHE["cnt"](dst)


def _seg_mean_sc(table, packed, inv):
    if "seg" not in _SC_CACHE:
        _SC_CACHE["seg"] = functools.partial(
            pl.kernel,
            mesh=_sc_mesh(),
            out_type=jax.ShapeDtypeStruct((2 * NPAD, DH), jnp.float32),
            scratch_types=[
                pltpu.VMEM((2 * CH,), jnp.int32),
                pltpu.VMEM((2 * CH,), jnp.int32),
                pltpu.VMEM((CH,), jnp.int32),
                pltpu.VMEM((CH,), jnp.int32),
                pltpu.VMEM((CH, DH), jnp.float32),
                pltpu.VMEM((CH,), jnp.int32),
                pltpu.VMEM((CH,), jnp.int32),
                pltpu.VMEM((CH, DH), jnp.float32),
                pltpu.VMEM((WCH, DH), jnp.float32),
                pltpu.VMEM((WCH, LANES), jnp.float32),
                pltpu.VMEM_SHARED((NPAD, DH), jnp.float32),
                pltpu.SemaphoreType.DMA,
                pltpu.SemaphoreType.DMA,
                pltpu.SemaphoreType.DMA,
                pltpu.SemaphoreType.DMA,
                pltpu.SemaphoreType.DMA,
                pltpu.SemaphoreType.DMA,
            ],
        )(_seg_body)
    return _SC_CACHE["seg"](table, packed, inv)


def _layer1_tc(a0, a1, x_split, Wl, bl, Wr0, Wr1, W2l, rows_per_blk=1000):
    grid = (N // rows_per_blk,)

    def body(a0_r, a1_r, xs_r, Wl_r, bl_r, Wr0_r, Wr1_r, W2_r, h_ref, y2_ref):
        mean = jnp.concatenate([a0_r[...], a1_r[...]], axis=1)
        pre = (_dot(mean, Wl_r[...]) + bl_r[...]
               + _dot(xs_r[0], Wr0_r[...]) + _dot(xs_r[1], Wr1_r[...]))
        h = _lk(pre)
        h_ref[...] = h
        y2 = _dot(h, W2_r[...])
        y2_ref[0] = y2[:, :DH]
        y2_ref[1] = y2[:, DH:]

    R = rows_per_blk
    return pl.pallas_call(
        body,
        grid=grid,
        in_specs=[
            pl.BlockSpec((R, DH), lambda i: (i, 0)),
            pl.BlockSpec((R, DH), lambda i: (i, 0)),
            pl.BlockSpec((2, R, DH), lambda i: (0, i, 0)),
            pl.BlockSpec(Wl.shape, lambda i: (0, 0)),
            pl.BlockSpec(bl.shape, lambda i: (0, 0)),
            pl.BlockSpec(Wr0.shape, lambda i: (0, 0)),
            pl.BlockSpec(Wr1.shape, lambda i: (0, 0)),
            pl.BlockSpec(W2l.shape, lambda i: (0, 0)),
        ],
        out_specs=[
            pl.BlockSpec((R, HID), lambda i: (i, 0)),
            pl.BlockSpec((2, R, DH), lambda i: (0, i, 0)),
        ],
        out_shape=[
            jax.ShapeDtypeStruct((N, HID), jnp.float32),
            jax.ShapeDtypeStruct((2, N, DH), jnp.float32),
        ],
    )(a0, a1, x_split, Wl, bl, Wr0, Wr1, W2l)


def _layer2_tc(b0, b1, h, Wr, bl, rows_per_blk=1000):
    grid = (N // rows_per_blk,)

    def body(b0_r, b1_r, h_r, Wr_r, bl_r, out_ref):
        mean = jnp.concatenate([b0_r[...], b1_r[...]], axis=1)
        out_ref[...] = mean + bl_r[...] + _dot(h_r[...], Wr_r[...])

    R = rows_per_blk
    return pl.pallas_call(
        body,
        grid=grid,
        in_specs=[
            pl.BlockSpec((R, DH), lambda i: (i, 0)),
            pl.BlockSpec((R, DH), lambda i: (i, 0)),
            pl.BlockSpec((R, HID), lambda i: (i, 0)),
            pl.BlockSpec(Wr.shape, lambda i: (0, 0)),
            pl.BlockSpec(bl.shape, lambda i: (0, 0)),
        ],
        out_specs=pl.BlockSpec((R, D), lambda i: (i, 0)),
        out_shape=jax.ShapeDtypeStruct((N, D), jnp.float32),
    )(b0, b1, h, Wr, bl)


def _pad_edges(ei):
    pad = EPAD - E
    src = jnp.concatenate([ei[0].astype(jnp.int32),
                           jnp.zeros((pad,), jnp.int32)])
    dst = jnp.concatenate([ei[1].astype(jnp.int32),
                           jnp.full((pad,), N, jnp.int32)])
    srcs2 = jnp.stack([src, src + N])
    nchunk = EPW // CH
    src_r = srcs2.reshape(2, NSUB, nchunk, CH)
    dst_r = jnp.broadcast_to(dst.reshape(1, NSUB, nchunk, CH),
                             (2, NSUB, nchunk, CH))
    packed = jnp.concatenate([src_r, dst_r], axis=3).reshape(-1)
    return packed, dst


def _halves(seg_out):
    return seg_out[:N], seg_out[NPAD:NPAD + N]


def kernel(m_emb_feat, m_sim_feat, m_ass_feat, d_sim_feat, d_ass_feat,
           ei_md, ei_dm,
           fu_m_emb_W, fu_m_emb_b, fu_m_sim_W, fu_m_sim_b,
           fu_m_ass_W, fu_m_ass_b, fu_d_sim_W, fu_d_sim_b,
           fu_d_ass_W, fu_d_ass_b,
           m_emb_W, m_emb_b, m_sim_W, m_sim_b, m_ass_W, m_ass_b,
           d_sim_W, d_sim_b, d_ass_W, d_ass_b,
           w_m_emb, w_m_sim, w_m_ass, w_d_sim, w_d_ass,
           l1_md_Wl, l1_md_bl, l1_md_Wr,
           l2_md_Wl, l2_md_bl, l2_md_Wr,
           l1_dm_Wl, l1_dm_bl, l1_dm_Wr,
           l2_dm_Wl, l2_dm_bl, l2_dm_Wr):
    pm = [m_emb_W * w_m_emb, m_sim_W * w_m_sim, m_ass_W * w_m_ass]
    bm = (m_emb_b * w_m_emb + m_sim_b * w_m_sim
          + m_ass_b * w_m_ass).reshape(1, D)
    pd = [d_sim_W * w_d_sim, d_ass_W * w_d_ass]
    bd = (d_sim_b * w_d_sim + d_ass_b * w_d_ass).reshape(1, D)

    src_md, dst_md = _pad_edges(ei_md)
    src_dm, dst_dm = _pad_edges(ei_dm)

    invs = _cnt_sc(jnp.concatenate([dst_md, dst_dm]))
    inv_md = invs[:NPAD]
    inv_dm = invs[NPAD:]

    x_m = _features_tc(
        [m_emb_feat, m_sim_feat, m_ass_feat],
        [fu_m_emb_W, fu_m_sim_W, fu_m_ass_W],
        [fu_m_emb_b.reshape(1, -1), fu_m_sim_b.reshape(1, -1),
         fu_m_ass_b.reshape(1, -1)],
        pm, bm)
    x_d = _features_tc(
        [d_sim_feat, d_ass_feat],
        [fu_d_sim_W, fu_d_ass_W],
        [fu_d_sim_b.reshape(1, -1), fu_d_ass_b.reshape(1, -1)],
        pd, bd)

    mean_md = _seg_mean_sc(x_m.reshape(2 * N, DH), src_md, inv_md)
    mean_dm = _seg_mean_sc(x_d.reshape(2 * N, DH), src_dm, inv_dm)

    a0_md, a1_md = _halves(mean_md)
    a0_dm, a1_dm = _halves(mean_dm)

    h_d, y2_dm = _layer1_tc(a0_md, a1_md, x_d,
                            l1_md_Wl, l1_md_bl.reshape(1, -1),
                            l1_md_Wr[:DH], l1_md_Wr[DH:], l2_dm_Wl)
    h_m, y2_md = _layer1_tc(a0_dm, a1_dm, x_m,
                            l1_dm_Wl, l1_dm_bl.reshape(1, -1),
                            l1_dm_Wr[:DH], l1_dm_Wr[DH:], l2_md_Wl)

    mean2_md = _seg_mean_sc(y2_md.reshape(2 * N, DH), src_md, inv_md)
    mean2_dm = _seg_mean_sc(y2_dm.reshape(2 * N, DH), src_dm, inv_dm)

    b0_md, b1_md = _halves(mean2_md)
    b0_dm, b1_dm = _halves(mean2_dm)

    out_d = _layer2_tc(b0_md, b1_md, h_d, l2_md_Wr, l2_md_bl.reshape(1, -1))
    out_m = _layer2_tc(b0_dm, b1_dm, h_m, l2_dm_Wr, l2_dm_bl.reshape(1, -1))
    return jnp.concatenate([out_m, out_d], axis=0)


# speedup vs baseline: 4.6728x; 1.0325x over previous
import functools

import jax
import jax.numpy as jnp
from jax import lax
from jax.experimental import pallas as pl
from jax.experimental.pallas import tpu as pltpu
from jax.experimental.pallas import tpu_sc as plsc

N = 10000
E = 160000
D = 256
DH = 128
HID = 512
SLOPE = 0.2

LANES = 16
NSUB = 16
CH = 128
EPW = 10112
EPAD = NSUB * EPW
NPAD = 10240
STRIPE = NPAD // NSUB
WCH = 32


def _lk(x):
    return jnp.where(x >= 0, x, SLOPE * x)


def _dot(a, b):
    return jnp.dot(a, b, preferred_element_type=jnp.float32)


def _features_tc(feats, fu_ws, fu_bs, proj_ws, bsum, rows_per_blk=1000):
    nf = len(feats)
    grid = (N // rows_per_blk,)

    def body(*refs):
        frefs = refs[:nf]
        fw = refs[nf:2 * nf]
        fb = refs[2 * nf:3 * nf]
        pw = refs[3 * nf:4 * nf]
        bsum_r = refs[4 * nf]
        out_ref = refs[4 * nf + 1]
        acc = None
        for i in range(nf):
            u = _lk(_dot(frefs[i][...], fw[i][...]) + fb[i][...])
            t = _dot(u, pw[i][...])
            acc = t if acc is None else acc + t
        x = acc + bsum_r[...]
        out_ref[0] = x[:, :DH]
        out_ref[1] = x[:, DH:]

    in_specs = []
    for f in feats:
        d = f.shape[1]
        in_specs.append(pl.BlockSpec((rows_per_blk, d), lambda i: (i, 0)))
    for w in fu_ws:
        in_specs.append(pl.BlockSpec(w.shape, lambda i: (0, 0)))
    for b in fu_bs:
        in_specs.append(pl.BlockSpec(b.shape, lambda i: (0, 0)))
    for w in proj_ws:
        in_specs.append(pl.BlockSpec(w.shape, lambda i: (0, 0)))
    in_specs.append(pl.BlockSpec(bsum.shape, lambda i: (0, 0)))

    return pl.pallas_call(
        body,
        grid=grid,
        in_specs=in_specs,
        out_specs=pl.BlockSpec((2, rows_per_blk, DH), lambda i: (0, i, 0)),
        out_shape=jax.ShapeDtypeStruct((2, N, DH), jnp.float32),
    )(*feats, *fu_ws, *fu_bs, *proj_ws, bsum)


def _cnt_body(dst_h, out_h, dst0_v, dst1_v, ones_v, stage_v, cstage_v,
              cnt_sh, sem0, sem1, semi0, semi1):
    c = lax.axis_index("c")
    s = lax.axis_index("s")
    zero16 = jnp.zeros((LANES,), jnp.float32)
    one16 = jnp.ones((LANES,), jnp.float32)

    def zrow(r, carry):
        for j in range(DH // LANES):
            stage_v[r, pl.ds(j * LANES, LANES)] = zero16
        return carry

    lax.fori_loop(0, WCH, zrow, 0)

    row0 = s * STRIPE

    def zchunk(k, carry):
        pltpu.sync_copy(stage_v, cnt_sh.at[pl.ds(row0 + k * WCH, WCH)])
        return carry

    lax.fori_loop(0, STRIPE // WCH, zchunk, 0)

    def orow(r, carry):
        for j in range(DH // LANES):
            ones_v[r, pl.ds(j * LANES, LANES)] = one16
        return carry

    lax.fori_loop(0, CH, orow, 0)
    plsc.subcore_barrier()

    ebase = c * EPAD + s * EPW
    npair = (EPW // CH) // 2
    bufs = ((dst0_v, sem0, semi0), (dst1_v, sem1, semi1))

    def islice(j):
        return dst_h.at[pl.ds(ebase + j * CH, CH)]

    def wait_scat(buf):
        pltpu.make_async_copy(ones_v, cnt_sh.at[buf[0]], buf[1]).wait()

    pltpu.async_copy(islice(0), dst0_v, semi0)

    def pair(i2, carry):
        for p in range(2):
            j = i2 * 2 + p
            cur, nxt = bufs[p], bufs[1 - p]
            pltpu.make_async_copy(islice(j), cur[0], cur[2]).wait()
            pltpu.async_copy(ones_v, cnt_sh.at[cur[0]], cur[1], add=True)
            if p == 0:
                @pl.when(i2 > 0)
                def _():
                    wait_scat(nxt)
            else:
                wait_scat(nxt)
            pltpu.async_copy(islice(j + 1), nxt[0], nxt[2])
        return carry

    lax.fori_loop(0, npair, pair, 0)
    pltpu.make_async_copy(islice(2 * npair), dst0_v, semi0).wait()
    pltpu.async_copy(ones_v, cnt_sh.at[dst0_v], sem0, add=True)
    wait_scat(bufs[1])
    wait_scat(bufs[0])
    plsc.subcore_barrier()

    def wchunk(k, carry):
        r0 = row0 + k * WCH
        pltpu.sync_copy(cnt_sh.at[pl.ds(r0, WCH)], stage_v)

        def irow(r, carry2):
            cstage_v[r, :] = 1.0 / jnp.maximum(stage_v[r, pl.ds(0, LANES)],
                                               1.0)
            return carry2

        lax.fori_loop(0, WCH, irow, 0)
        pltpu.sync_copy(cstage_v, out_h.at[pl.ds(c * NPAD + r0, WCH)])
        return carry

    lax.fori_loop(0, STRIPE // WCH, wchunk, 0)


def _seg_body(table_h, packed_h, inv_h, out_h,
              big0_v, big1_v, src0_v, dst0_v, rows0_v,
              src1_v, dst1_v, rows1_v,
              stage_v, cstage_v, acc_sh,
              semg0, sems0, semg1, sems1, semi0, semi1):
    c = lax.axis_index("c")
    s = lax.axis_index("s")
    zero16 = jnp.zeros((LANES,), jnp.float32)

    def zrow(r, carry):
        for j in range(DH // LANES):
            stage_v[r, pl.ds(j * LANES, LANES)] = zero16
        return carry

    lax.fori_loop(0, WCH, zrow, 0)

    row0 = s * STRIPE

    def zchunk(k, carry):
        pltpu.sync_copy(stage_v, acc_sh.at[pl.ds(row0 + k * WCH, WCH)])
        return carry

    lax.fori_loop(0, STRIPE // WCH, zchunk, 0)
    plsc.subcore_barrier()

    nchunk = EPW // CH
    npair = nchunk // 2
    pbase = (c * NSUB + s) * (nchunk * 2 * CH)
    bufs = ((src0_v, dst0_v, rows0_v, semg0, sems0),
            (src1_v, dst1_v, rows1_v, semg1, sems1))
    bigs = ((big0_v, semi0), (big1_v, semi1))

    def pslice(j):
        return packed_h.at[pl.ds(pbase + j * 2 * CH, 2 * CH)]

    def unpack(big, buf):
        for q in range(CH // LANES):
            sl = pl.ds(q * LANES, LANES)
            buf[0][sl] = big[0][pl.ds(q * LANES, LANES)]
            buf[1][sl] = big[0][pl.ds(CH + q * LANES, LANES)]

    def wait_scat(buf):
        pltpu.make_async_copy(buf[2], acc_sh.at[buf[1]], buf[4]).wait()

    pltpu.async_copy(pslice(0), big0_v, semi0)
    pltpu.make_async_copy(pslice(0), big0_v, semi0).wait()
    unpack(bigs[0], bufs[0])
    pltpu.async_copy(pslice(1), big1_v, semi1)
    pltpu.async_copy(table_h.at[src0_v], rows0_v, semg0)

    def pair(i2, carry):
        for p in range(2):
            j = i2 * 2 + p
            cur, nxt = bufs[p], bufs[1 - p]
            bigc, bign = bigs[p], bigs[1 - p]
            if p == 0:
                @pl.when(i2 > 0)
                def _():
                    wait_scat(nxt)
            else:
                wait_scat(nxt)
            pltpu.make_async_copy(pslice(j + 1), bign[0], bign[1]).wait()
            unpack(bign, nxt)
            if p == 0:
                pltpu.async_copy(pslice(j + 2), bigc[0], bigc[1])
            else:
                @pl.when(i2 < npair - 1)
                def _():
                    pltpu.async_copy(pslice(j + 2), bigc[0], bigc[1])
            pltpu.async_copy(table_h.at[nxt[0]], nxt[2], nxt[3])
            pltpu.make_async_copy(table_h.at[cur[0]], cur[2], cur[3]).wait()
            pltpu.async_copy(cur[2], acc_sh.at[cur[1]], cur[4], add=True)
        return carry

    lax.fori_loop(0, npair, pair, 0)
    pltpu.make_async_copy(table_h.at[src0_v], rows0_v, semg0).wait()
    pltpu.async_copy(rows0_v, acc_sh.at[dst0_v], sems0, add=True)
    wait_scat(bufs[0])
    wait_scat(bufs[1])
    plsc.subcore_barrier()

    def wchunk(k, carry):
        r0 = row0 + k * WCH
        pltpu.sync_copy(acc_sh.at[pl.ds(r0, WCH)], stage_v)
        pltpu.sync_copy(inv_h.at[pl.ds(r0, WCH)], cstage_v)

        def div_row(r, carry2):
            inv = cstage_v[r, :]
            for j in range(DH // LANES):
                sl = pl.ds(j * LANES, LANES)
                stage_v[r, sl] = stage_v[r, sl] * inv
            return carry2

        lax.fori_loop(0, WCH, div_row, 0)
        pltpu.sync_copy(stage_v, out_h.at[pl.ds(c * NPAD + r0, WCH)])
        return carry

    lax.fori_loop(0, STRIPE // WCH, wchunk, 0)


_SC_CACHE = {}


def _sc_mesh():
    return plsc.VectorSubcoreMesh(core_axis_name="c", subcore_axis_name="s")


def _cnt_sc(dst):
    if "cnt" not in _SC_CACHE:
        _SC_CACHE["cnt"] = functools.partial(
            pl.kernel,
            mesh=_sc_mesh(),
            out_type=jax.ShapeDtypeStruct((2 * NPAD, LANES), jnp.float32),
            scratch_types=[
                pltpu.VMEM((CH,), jnp.int32),
                pltpu.VMEM((CH,), jnp.int32),
                pltpu.VMEM((CH, DH), jnp.float32),
                pltpu.VMEM((WCH, DH), jnp.float32),
                pltpu.VMEM((WCH, LANES), jnp.float32),
                pltpu.VMEM_SHARED((NPAD, DH), jnp.float32),
                pltpu.SemaphoreType.DMA,
                pltpu.SemaphoreType.DMA,
                pltpu.SemaphoreType.DMA,
                pltpu.SemaphoreType.DMA,
            ],
        )(_cnt_body)
    return _SC_CACHE["cnt"](dst)


def _seg_mean_sc(table, packed, inv):
    if "seg" not in _SC_CACHE:
        _SC_CACHE["seg"] = functools.partial(
            pl.kernel,
            mesh=_sc_mesh(),
            out_type=jax.ShapeDtypeStruct((2 * NPAD, DH), jnp.float32),
            scratch_types=[
                pltpu.VMEM((2 * CH,), jnp.int32),
                pltpu.VMEM((2 * CH,), jnp.int32),
                pltpu.VMEM((CH,), jnp.int32),
                pltpu.VMEM((CH,), jnp.int32),
                pltpu.VMEM((CH, DH), jnp.float32),
                pltpu.VMEM((CH,), jnp.int32),
                pltpu.VMEM((CH,), jnp.int32),
                pltpu.VMEM((CH, DH), jnp.float32),
                pltpu.VMEM((WCH, DH), jnp.float32),
                pltpu.VMEM((WCH, LANES), jnp.float32),
                pltpu.VMEM_SHARED((NPAD, DH), jnp.float32),
                pltpu.SemaphoreType.DMA,
                pltpu.SemaphoreType.DMA,
                pltpu.SemaphoreType.DMA,
                pltpu.SemaphoreType.DMA,
                pltpu.SemaphoreType.DMA,
                pltpu.SemaphoreType.DMA,
            ],
        )(_seg_body)
    return _SC_CACHE["seg"](table, packed, inv)


def _layer1_tc(a0, a1, x_split, Wl, bl, Wr0, Wr1, W2l, rows_per_blk=1000):
    grid = (N // rows_per_blk,)

    def body(a0_r, a1_r, xs_r, Wl_r, bl_r, Wr0_r, Wr1_r, W2_r, h_ref, y2_ref):
        mean = jnp.concatenate([a0_r[...], a1_r[...]], axis=1)
        pre = (_dot(mean, Wl_r[...]) + bl_r[...]
               + _dot(xs_r[0], Wr0_r[...]) + _dot(xs_r[1], Wr1_r[...]))
        h = _lk(pre)
        h_ref[...] = h
        y2 = _dot(h, W2_r[...])
        y2_ref[0] = y2[:, :DH]
        y2_ref[1] = y2[:, DH:]

    R = rows_per_blk
    return pl.pallas_call(
        body,
        grid=grid,
        in_specs=[
            pl.BlockSpec((R, DH), lambda i: (i, 0)),
            pl.BlockSpec((R, DH), lambda i: (i, 0)),
            pl.BlockSpec((2, R, DH), lambda i: (0, i, 0)),
            pl.BlockSpec(Wl.shape, lambda i: (0, 0)),
            pl.BlockSpec(bl.shape, lambda i: (0, 0)),
            pl.BlockSpec(Wr0.shape, lambda i: (0, 0)),
            pl.BlockSpec(Wr1.shape, lambda i: (0, 0)),
            pl.BlockSpec(W2l.shape, lambda i: (0, 0)),
        ],
        out_specs=[
            pl.BlockSpec((R, HID), lambda i: (i, 0)),
            pl.BlockSpec((2, R, DH), lambda i: (0, i, 0)),
        ],
        out_shape=[
            jax.ShapeDtypeStruct((N, HID), jnp.float32),
            jax.ShapeDtypeStruct((2, N, DH), jnp.float32),
        ],
    )(a0, a1, x_split, Wl, bl, Wr0, Wr1, W2l)


def _layer2_tc(b0, b1, h, Wr, bl, rows_per_blk=1000):
    grid = (N // rows_per_blk,)

    def body(b0_r, b1_r, h_r, Wr_r, bl_r, out_ref):
        mean = jnp.concatenate([b0_r[...], b1_r[...]], axis=1)
        out_ref[...] = mean + bl_r[...] + _dot(h_r[...], Wr_r[...])

    R = rows_per_blk
    return pl.pallas_call(
        body,
        grid=grid,
        in_specs=[
            pl.BlockSpec((R, DH), lambda i: (i, 0)),
            pl.BlockSpec((R, DH), lambda i: (i, 0)),
            pl.BlockSpec((R, HID), lambda i: (i, 0)),
            pl.BlockSpec(Wr.shape, lambda i: (0, 0)),
            pl.BlockSpec(bl.shape, lambda i: (0, 0)),
        ],
        out_specs=pl.BlockSpec((R, D), lambda i: (i, 0)),
        out_shape=jax.ShapeDtypeStruct((N, D), jnp.float32),
    )(b0, b1, h, Wr, bl)


def _pad_edges(ei):
    pad = EPAD - E
    src = jnp.concatenate([ei[0].astype(jnp.int32),
                           jnp.zeros((pad,), jnp.int32)])
    dst = jnp.concatenate([ei[1].astype(jnp.int32),
                           jnp.full((pad,), N, jnp.int32)])
    srcs2 = jnp.stack([src, src + N])
    nchunk = EPW // CH
    src_r = srcs2.reshape(2, NSUB, nchunk, CH)
    dst_r = jnp.broadcast_to(dst.reshape(1, NSUB, nchunk, CH),
                             (2, NSUB, nchunk, CH))
    packed = jnp.concatenate([src_r, dst_r], axis=3).reshape(-1)
    return packed, dst


def _halves(seg_out):
    return seg_out[:N], seg_out[NPAD:NPAD + N]


def kernel(m_emb_feat, m_sim_feat, m_ass_feat, d_sim_feat, d_ass_feat,
           ei_md, ei_dm,
           fu_m_emb_W, fu_m_emb_b, fu_m_sim_W, fu_m_sim_b,
           fu_m_ass_W, fu_m_ass_b, fu_d_sim_W, fu_d_sim_b,
           fu_d_ass_W, fu_d_ass_b,
           m_emb_W, m_emb_b, m_sim_W, m_sim_b, m_ass_W, m_ass_b,
           d_sim_W, d_sim_b, d_ass_W, d_ass_b,
           w_m_emb, w_m_sim, w_m_ass, w_d_sim, w_d_ass,
           l1_md_Wl, l1_md_bl, l1_md_Wr,
           l2_md_Wl, l2_md_bl, l2_md_Wr,
           l1_dm_Wl, l1_dm_bl, l1_dm_Wr,
           l2_dm_Wl, l2_dm_bl, l2_dm_Wr):
    pm = [m_emb_W * w_m_emb, m_sim_W * w_m_sim, m_ass_W * w_m_ass]
    bm = (m_emb_b * w_m_emb + m_sim_b * w_m_sim
          + m_ass_b * w_m_ass).reshape(1, D)
    pd = [d_sim_W * w_d_sim, d_ass_W * w_d_ass]
    bd = (d_sim_b * w_d_sim + d_ass_b * w_d_ass).reshape(1, D)

    src_md, dst_md = _pad_edges(ei_md)
    src_dm, dst_dm = _pad_edges(ei_dm)

    invs = _cnt_sc(jnp.concatenate([dst_md, dst_dm]))
    inv_md = invs[:NPAD]
    inv_dm = invs[NPAD:]

    x_m = _features_tc(
        [m_emb_feat, m_sim_feat, m_ass_feat],
        [fu_m_emb_W, fu_m_sim_W, fu_m_ass_W],
        [fu_m_emb_b.reshape(1, -1), fu_m_sim_b.reshape(1, -1),
         fu_m_ass_b.reshape(1, -1)],
        pm, bm)
    x_d = _features_tc(
        [d_sim_feat, d_ass_feat],
        [fu_d_sim_W, fu_d_ass_W],
        [fu_d_sim_b.reshape(1, -1), fu_d_ass_b.reshape(1, -1)],
        pd, bd)

    mean_md = _seg_mean_sc(x_m.reshape(2 * N, DH), src_md, inv_md)
    mean_dm = _seg_mean_sc(x_d.reshape(2 * N, DH), src_dm, inv_dm)

    a0_md, a1_md = _halves(mean_md)
    a0_dm, a1_dm = _halves(mean_dm)

    h_d, y2_dm = _layer1_tc(a0_md, a1_md, x_d,
                            l1_md_Wl, l1_md_bl.reshape(1, -1),
                            l1_md_Wr[:DH], l1_md_Wr[DH:], l2_dm_Wl)
    h_m, y2_md = _layer1_tc(a0_dm, a1_dm, x_m,
                            l1_dm_Wl, l1_dm_bl.reshape(1, -1),
                            l1_dm_Wr[:DH], l1_dm_Wr[DH:], l2_md_Wl)

    mean2_md = _seg_mean_sc(y2_md.reshape(2 * N, DH), src_md, inv_md)
    mean2_dm = _seg_mean_sc(y2_dm.reshape(2 * N, DH), src_dm, inv_dm)

    b0_md, b1_md = _halves(mean2_md)
    b0_dm, b1_dm = _halves(mean2_dm)

    out_d = _layer2_tc(b0_md, b1_md, h_d, l2_md_Wr, l2_md_bl.reshape(1, -1))
    out_m = _layer2_tc(b0_dm, b1_dm, h_m, l2_dm_Wr, l2_dm_bl.reshape(1, -1))
    return jnp.concatenate([out_m, out_d], axis=0)
